# Initial kernel scaffold; baseline (speedup 1.0000x reference)
import jax
import jax.numpy as jnp
from jax.experimental import pallas as pl


def kernel(x_desikan, edge_index_desikan, batch_desikan, x_destrieux, edge_index_destrieux, batch_destrieux, x_fuzzy, edge_index_fuzzy, batch_fuzzy, demographic, W1_des, b1_des, W2_des, b2_des, W1_det, b1_det, W2_det, b2_det, W1_fuz, b1_fuz, W2_fuz, b2_fuz, fc1_W, fc1_b, fc2_W, fc2_b, fc3_W, fc3_b):
    raise NotImplementedError("write your pallas kernel here")


# speedup vs baseline: 16.3607x; 16.3607x over previous
import functools

import jax
import jax.numpy as jnp
import numpy as np
from jax import lax
from jax.experimental import pallas as pl
from jax.experimental.pallas import tpu as pltpu
from jax.experimental.pallas import tpu_sc as plsc

N = 10000
B = 64
H = 128
OUT = 64
NC = 2
NS = 16
NW = NC * NS
CH = 128
DUM = 512
NPAD = 10752
RS = NPAD // NS
E_ALIGN = NW * CH * 2
DW = 16

@functools.cache
def _mesh():
    return plsc.VectorSubcoreMesh(core_axis_name="c", subcore_axis_name="s",
                                  num_cores=NC, num_subcores=NS)


def _pad_edges(ei, e_pad):
    e = ei.shape[1]
    k = jnp.arange(e_pad - e, dtype=jnp.int32)
    src = jnp.concatenate([ei[0], k % np.int32(N)])
    dst = jnp.concatenate([ei[1], np.int32(N) + (k % np.int32(DUM))])
    return src, dst


def _deg_body(npws, d0, d1, d2, z_ref, ones_ref, o0, o1, o2,
              acc, ones_v, idx_v):
    c = lax.axis_index("c")
    s = lax.axis_index("s")
    w = s * NC + c
    pltpu.sync_copy(ones_ref, ones_v)
    for dst_ref, out_ref, npw in zip((d0, d1, d2), (o0, o1, o2), npws):
        pltpu.sync_copy(z_ref.at[pl.ds(s * RS, RS)], acc.at[pl.ds(s * RS, RS)])
        plsc.subcore_barrier()

        def body(i, _):
            base = (w * npw + i) * CH
            pltpu.sync_copy(dst_ref.at[pl.ds(base, CH)], idx_v)
            pltpu.sync_copy(ones_v, acc.at[idx_v], add=True)
            return 0

        lax.fori_loop(0, npw, body, 0)
        plsc.subcore_barrier()
        pltpu.sync_copy(acc.at[pl.ds(s * RS, RS)],
                        out_ref.at[pl.ds(c * NPAD + s * RS, RS)])
        plsc.subcore_barrier()


def _sc_degrees(dsts, npws):
    z = jnp.zeros((NPAD, DW), jnp.float32)
    ones = jnp.ones((CH, DW), jnp.float32)
    out_t = [jax.ShapeDtypeStruct((2 * NPAD, DW), jnp.float32)] * 3
    fn = pl.kernel(
        functools.partial(_deg_body, tuple(npws)),
        out_type=out_t,
        mesh=_mesh(),
        scratch_types=[
            pltpu.VMEM_SHARED((NPAD, DW), jnp.float32),
            pltpu.VMEM((CH, DW), jnp.float32),
            pltpu.VMEM((CH,), jnp.int32),
        ],
        compiler_params=pltpu.CompilerParams(use_tc_tiling_on_sc=False),
        name="sc_degrees",
    )
    return fn(*dsts, z, ones)


def _agg_body(npws, hd, h0, h1, h2, s0, s1, s2, d0, d1, d2, z_ref,
              o0, o1, o2, acc, idxs, idxd, rows, sem0, sem1):
    c = lax.axis_index("c")
    s = lax.axis_index("s")
    w = s * NC + c
    sems = (sem0, sem1)
    for h_ref, src_ref, dst_ref, out_ref, npw in zip(
            (h0, h1, h2), (s0, s1, s2), (d0, d1, d2), (o0, o1, o2), npws):
        pltpu.sync_copy(z_ref.at[pl.ds(s * RS, RS)], acc.at[pl.ds(s * RS, RS)])
        plsc.subcore_barrier()

        base0 = w * npw * CH
        pltpu.sync_copy(src_ref.at[pl.ds(base0, CH)], idxs.at[0])
        pltpu.sync_copy(dst_ref.at[pl.ds(base0, CH)], idxd.at[0])
        pltpu.async_copy(h_ref.at[idxs.at[0]], rows.at[0], sems[0])

        def body(i, _):
            for b in (0, 1):
                k = 2 * i + b
                nb = 1 - b
                pltpu.make_async_copy(
                    h_ref.at[idxs.at[b]], rows.at[b], sems[b]).wait()

                @pl.when(k + 1 < npw)
                def _():
                    nbase = (w * npw + k + 1) * CH
                    pltpu.sync_copy(src_ref.at[pl.ds(nbase, CH)], idxs.at[nb])
                    pltpu.sync_copy(dst_ref.at[pl.ds(nbase, CH)], idxd.at[nb])
                    pltpu.async_copy(h_ref.at[idxs.at[nb]], rows.at[nb],
                                     sems[nb])

                pltpu.sync_copy(rows.at[b], acc.at[idxd.at[b]], add=True)
            return 0

        lax.fori_loop(0, npw // 2, body, 0)
        plsc.subcore_barrier()
        pltpu.sync_copy(acc.at[pl.ds(s * RS, RS)],
                        out_ref.at[pl.ds(c * NPAD + s * RS, RS)])
        plsc.subcore_barrier()


def _sc_aggregate(hs, srcs, dsts, npws, hd):
    z = jnp.zeros((NPAD, hd), jnp.float32)
    out_t = [jax.ShapeDtypeStruct((2 * NPAD, hd), jnp.float32)] * 3
    fn = pl.kernel(
        functools.partial(_agg_body, tuple(npws), hd),
        out_type=out_t,
        mesh=_mesh(),
        scratch_types=[
            pltpu.VMEM_SHARED((NPAD, hd), jnp.float32),
            pltpu.VMEM((2, CH), jnp.int32),
            pltpu.VMEM((2, CH), jnp.int32),
            pltpu.VMEM((2, CH, hd), jnp.float32),
            pltpu.SemaphoreType.DMA,
            pltpu.SemaphoreType.DMA,
        ],
        compiler_params=pltpu.CompilerParams(use_tc_tiling_on_sc=(hd == H)),
        name=f"sc_gcn_agg_{hd}",
    )
    return fn(*hs, *srcs, *dsts, z)


_BLK = 1000
_G = N // _BLK
_DOT = dict(preferred_element_type=jnp.float32,
            precision=jax.lax.Precision.HIGHEST)


def _tc1_body(x_ref, w_ref, degp_ref, hp_ref, dinv_ref):
    deg = degp_ref[0, :, 0:1] + degp_ref[1, :, 0:1] + 1.0
    dinv = lax.rsqrt(deg)
    h = lax.dot_general(x_ref[...], w_ref[...], (((1,), (0,)), ((), ())),
                        **_DOT)
    hp_ref[...] = h * dinv
    dinv_ref[...] = dinv


def _tc1(x, w1, degp):
    d = x.shape[1]
    return pl.pallas_call(
        _tc1_body,
        grid=(_G,),
        in_specs=[
            pl.BlockSpec((_BLK, d), lambda i: (i, 0)),
            pl.BlockSpec((d, H), lambda i: (0, 0)),
            pl.BlockSpec((2, _BLK, DW), lambda i: (0, i, 0)),
        ],
        out_specs=[
            pl.BlockSpec((_BLK, H), lambda i: (i, 0)),
            pl.BlockSpec((_BLK, 1), lambda i: (i, 0)),
        ],
        out_shape=[
            jax.ShapeDtypeStruct((N, H), jnp.float32),
            jax.ShapeDtypeStruct((N, 1), jnp.float32),
        ],
    )(x, w1, degp)


def _tc2_body(sp_ref, hp_ref, dinv_ref, b1_ref, w2_ref, out_ref):
    dinv = dinv_ref[...]
    y = (sp_ref[0] + sp_ref[1] + hp_ref[...]) * dinv + b1_ref[...]
    y = jnp.maximum(y, 0.0)
    h2 = lax.dot_general(y, w2_ref[...], (((1,), (0,)), ((), ())), **_DOT)
    out_ref[...] = h2 * dinv


def _tc2(sp, hp, dinv, b1, w2):
    return pl.pallas_call(
        _tc2_body,
        grid=(_G,),
        in_specs=[
            pl.BlockSpec((2, _BLK, H), lambda i: (0, i, 0)),
            pl.BlockSpec((_BLK, H), lambda i: (i, 0)),
            pl.BlockSpec((_BLK, 1), lambda i: (i, 0)),
            pl.BlockSpec((1, H), lambda i: (0, 0)),
            pl.BlockSpec((H, OUT), lambda i: (0, 0)),
        ],
        out_specs=pl.BlockSpec((_BLK, OUT), lambda i: (i, 0)),
        out_shape=jax.ShapeDtypeStruct((N, OUT), jnp.float32),
    )(sp, hp, dinv, b1, w2)


def _tc3_body(sp0, hp0, di0, bb0, bt0,
              sp1, hp1, di1, bb1, bt1,
              sp2, hp2, di2, bb2, bt2,
              demo_ref, f1w, f1b, f2w, f2b, f3w, f3b,
              out_ref, pooled, counts):
    i = pl.program_id(0)

    @pl.when(i == 0)
    def _():
        pooled[...] = jnp.zeros_like(pooled)
        counts[...] = jnp.zeros_like(counts)

    ones_col = jnp.ones((_BLK, 1), jnp.float32)
    for b, (sp, hp, di, bb, bt) in enumerate((
            (sp0, hp0, di0, bb0, bt0),
            (sp1, hp1, di1, bb1, bt1),
            (sp2, hp2, di2, bb2, bt2))):
        y = (sp[0] + sp[1] + hp[...]) * di[...] + bb[...]
        y = jnp.maximum(y, 0.0)
        gids = lax.broadcasted_iota(jnp.int32, (B, _BLK), 0)
        m = (gids == bt[0]).astype(jnp.float32)
        pooled[:, b * OUT:(b + 1) * OUT] += lax.dot_general(
            m, y, (((1,), (0,)), ((), ())), **_DOT)
        counts[:, b:b + 1] += lax.dot_general(
            m, ones_col, (((1,), (0,)), ((), ())), **_DOT)

    @pl.when(i == _G - 1)
    def _():
        cnt = jnp.maximum(counts[...], 1.0)
        h = f1b[...]
        for b in range(3):
            p = pooled[:, b * OUT:(b + 1) * OUT] / cnt[:, b:b + 1]
            h = h + lax.dot_general(
                p, f1w[b * OUT:(b + 1) * OUT, :],
                (((1,), (0,)), ((), ())), **_DOT)
        h = h + lax.dot_general(demo_ref[...], f1w[3 * OUT:, :],
                                (((1,), (0,)), ((), ())), **_DOT)
        h = jnp.maximum(h, 0.0)
        h = jnp.maximum(lax.dot_general(h, f2w[...],
                                        (((1,), (0,)), ((), ())), **_DOT)
                        + f2b[...], 0.0)
        out_ref[...] = lax.dot_general(h, f3w[...],
                                       (((1,), (0,)), ((), ())), **_DOT) \
            + f3b[...]


def _tc3(branches, demo, f1w, f1b, f2w, f2b, f3w, f3b):
    in_arrays = []
    in_specs = []
    for sp, hp, dinv, b2, bat3 in branches:
        in_arrays += [sp, hp, dinv, b2, bat3]
        in_specs += [
            pl.BlockSpec((2, _BLK, OUT), lambda i: (0, i, 0)),
            pl.BlockSpec((_BLK, OUT), lambda i: (i, 0)),
            pl.BlockSpec((_BLK, 1), lambda i: (i, 0)),
            pl.BlockSpec((1, OUT), lambda i: (0, 0)),
            pl.BlockSpec((1, 1, _BLK), lambda i: (i, 0, 0)),
        ]
    in_arrays += [demo, f1w, f1b, f2w, f2b, f3w, f3b]
    in_specs += [
        pl.BlockSpec((B, 16), lambda i: (0, 0)),
        pl.BlockSpec((3 * OUT + 16, B), lambda i: (0, 0)),
        pl.BlockSpec((1, B), lambda i: (0, 0)),
        pl.BlockSpec((B, 32), lambda i: (0, 0)),
        pl.BlockSpec((1, 32), lambda i: (0, 0)),
        pl.BlockSpec((32, 2), lambda i: (0, 0)),
        pl.BlockSpec((1, 2), lambda i: (0, 0)),
    ]
    return pl.pallas_call(
        _tc3_body,
        grid=(_G,),
        in_specs=in_specs,
        out_specs=pl.BlockSpec((B, 2), lambda i: (0, 0)),
        out_shape=jax.ShapeDtypeStruct((B, 2), jnp.float32),
        scratch_shapes=[
            pltpu.VMEM((B, 3 * OUT), jnp.float32),
            pltpu.VMEM((B, 8), jnp.float32),
        ],
    )(*in_arrays)


def kernel(x_desikan, edge_index_desikan, batch_desikan,
           x_destrieux, edge_index_destrieux, batch_destrieux,
           x_fuzzy, edge_index_fuzzy, batch_fuzzy,
           demographic,
           W1_des, b1_des, W2_des, b2_des,
           W1_det, b1_det, W2_det, b2_det,
           W1_fuz, b1_fuz, W2_fuz, b2_fuz,
           fc1_W, fc1_b, fc2_W, fc2_b, fc3_W, fc3_b):
    xs = (x_desikan, x_destrieux, x_fuzzy)
    eis = (edge_index_desikan, edge_index_destrieux, edge_index_fuzzy)
    bats = (batch_desikan, batch_destrieux, batch_fuzzy)
    w1s, b1s = (W1_des, W1_det, W1_fuz), (b1_des, b1_det, b1_fuz)
    w2s, b2s = (W2_des, W2_det, W2_fuz), (b2_des, b2_det, b2_fuz)

    srcs, dsts, npws = [], [], []
    for ei in eis:
        e_pad = -(-ei.shape[1] // E_ALIGN) * E_ALIGN
        s, d = _pad_edges(ei, e_pad)
        srcs.append(s)
        dsts.append(d)
        npws.append(e_pad // (NW * CH))
    srcs, dsts, npws = tuple(srcs), tuple(dsts), tuple(npws)

    degps = _sc_degrees(dsts, npws)
    degps = [p.reshape(2, NPAD, DW) for p in degps]

    h1ps, dinvs = [], []
    for x, w1, degp in zip(xs, w1s, degps):
        hp, dinv = _tc1(x, w1, degp)
        h1ps.append(hp)
        dinvs.append(dinv)

    s1ps = _sc_aggregate(tuple(h1ps), srcs, dsts, npws, H)
    s1ps = [p.reshape(2, NPAD, H) for p in s1ps]

    h2ps = []
    for sp, hp, dinv, b1, w2 in zip(s1ps, h1ps, dinvs, b1s, w2s):
        h2ps.append(_tc2(sp, hp, dinv, b1.reshape(1, H), w2))

    s2ps = _sc_aggregate(tuple(h2ps), srcs, dsts, npws, OUT)
    s2ps = [p.reshape(2, NPAD, OUT) for p in s2ps]

    branches = []
    for sp, hp, dinv, b2, bat in zip(s2ps, h2ps, dinvs, b2s, bats):
        branches.append((sp, hp, dinv, b2.reshape(1, OUT),
                         bat.reshape(_G, 1, _BLK)))

    return _tc3(branches, demographic, fc1_W, fc1_b.reshape(1, B),
                fc2_W, fc2_b.reshape(1, 32), fc3_W, fc3_b.reshape(1, 2))


# speedup vs baseline: 25.2468x; 1.5431x over previous
import functools

import jax
import jax.numpy as jnp
import numpy as np
from jax import lax
from jax.experimental import pallas as pl
from jax.experimental.pallas import tpu as pltpu
from jax.experimental.pallas import tpu_sc as plsc

N = 10000
B = 64
H = 128
OUT = 64
NC = 2
NS = 16
NW = NC * NS
CH = 128
DUM = 512
NPAD = 10752
RS = NPAD // NS
QB = 8
E_ALIGN = NW * CH * QB
DW = 8

@functools.cache
def _mesh():
    return plsc.VectorSubcoreMesh(core_axis_name="c", subcore_axis_name="s",
                                  num_cores=NC, num_subcores=NS)


def _pad_edges(ei, e_pad):
    e = ei.shape[1]
    k = jnp.arange(e_pad - e, dtype=jnp.int32)
    src = jnp.concatenate([ei[0], k % np.int32(N)]).reshape(e_pad // CH, CH)
    dst = jnp.concatenate([ei[1], np.int32(N) + (k % np.int32(DUM))])
    return src, dst.reshape(e_pad // CH, CH)


def _deg_body(npws, d0, d1, d2, z_ref, ones_ref, o0, o1, o2,
              acc, ones_v, idxd, isem, ssem):
    c = lax.axis_index("c")
    s = lax.axis_index("s")
    w = s * NC + c
    pltpu.sync_copy(ones_ref, ones_v)
    for dst_ref, out_ref, npw in zip((d0, d1, d2), (o0, o1, o2), npws):
        nblk = npw // QB
        pltpu.sync_copy(z_ref.at[pl.ds(s * RS, RS)], acc.at[pl.ds(s * RS, RS)])
        plsc.subcore_barrier()
        row0 = w * npw
        pltpu.sync_copy(dst_ref.at[pl.ds(row0, QB)], idxd.at[0])

        def body(jb, _):
            jm = jb % 2
            jn = (jb + 1) % 2

            @pl.when(jb + 1 < nblk)
            def _():
                pltpu.async_copy(
                    dst_ref.at[pl.ds(row0 + (jb + 1) * QB, QB)],
                    idxd.at[jn], isem)

            for q in range(QB):
                pltpu.async_copy(ones_v, acc.at[idxd.at[jm, q]], ssem,
                                 add=True)
            for q in range(QB):
                pltpu.make_async_copy(ones_v, acc.at[idxd.at[jm, q]],
                                      ssem).wait()

            @pl.when(jb + 1 < nblk)
            def _():
                pltpu.make_async_copy(
                    dst_ref.at[pl.ds(row0, QB)], idxd.at[jn], isem).wait()
            return 0

        lax.fori_loop(0, nblk, body, 0)
        plsc.subcore_barrier()
        pltpu.sync_copy(acc.at[pl.ds(s * RS, RS)],
                        out_ref.at[pl.ds(c * NPAD + s * RS, RS)])
        plsc.subcore_barrier()


def _sc_degrees(dsts, npws):
    z = jnp.zeros((NPAD, DW), jnp.float32)
    ones = jnp.ones((CH, DW), jnp.float32)
    out_t = [jax.ShapeDtypeStruct((2 * NPAD, DW), jnp.float32)] * 3
    fn = pl.kernel(
        functools.partial(_deg_body, tuple(npws)),
        out_type=out_t,
        mesh=_mesh(),
        scratch_types=[
            pltpu.VMEM_SHARED((NPAD, DW), jnp.float32),
            pltpu.VMEM((CH, DW), jnp.float32),
            pltpu.VMEM((2, QB, CH), jnp.int32),
            pltpu.SemaphoreType.DMA,
            pltpu.SemaphoreType.DMA,
        ],
        compiler_params=pltpu.CompilerParams(use_tc_tiling_on_sc=False),
        name="sc_degrees",
    )
    return fn(*dsts, z, ones)


def _agg_body(npws, hd, h0, h1, h2, s0, s1, s2, d0, d1, d2, z_ref,
              o0, o1, o2, acc, idxs, idxd, rows, gs0, gs1, ss0, ss1, isem):
    c = lax.axis_index("c")
    s = lax.axis_index("s")
    w = s * NC + c
    gsems = (gs0, gs1)
    ssems = (ss0, ss1)
    for h_ref, src_ref, dst_ref, out_ref, npw in zip(
            (h0, h1, h2), (s0, s1, s2), (d0, d1, d2), (o0, o1, o2), npws):
        nblk = npw // QB
        pltpu.sync_copy(z_ref.at[pl.ds(s * RS, RS)], acc.at[pl.ds(s * RS, RS)])
        plsc.subcore_barrier()

        row0 = w * npw
        pltpu.sync_copy(src_ref.at[pl.ds(row0, QB)], idxs.at[0])
        pltpu.sync_copy(dst_ref.at[pl.ds(row0, QB)], idxd.at[0])
        pltpu.async_copy(h_ref.at[idxs.at[0, 0]], rows.at[0], gsems[0])

        def body(jb, _):
            jm = jb % 2
            jn = (jb + 1) % 2

            @pl.when(jb > 0)
            def _():
                pltpu.make_async_copy(
                    rows.at[1], acc.at[idxd.at[jn, QB - 1]], ssems[1]).wait()

            @pl.when(jb + 1 < nblk)
            def _():
                pltpu.async_copy(
                    src_ref.at[pl.ds(row0 + (jb + 1) * QB, QB)],
                    idxs.at[jn], isem)
                pltpu.async_copy(
                    dst_ref.at[pl.ds(row0 + (jb + 1) * QB, QB)],
                    idxd.at[jn], isem)

            for q in range(QB):
                b = q % 2
                nb = 1 - b
                pltpu.make_async_copy(
                    h_ref.at[idxs.at[jm, q]], rows.at[b], gsems[b]).wait()
                pltpu.async_copy(rows.at[b], acc.at[idxd.at[jm, q]],
                                 ssems[b], add=True)
                if 0 < q:
                    pltpu.make_async_copy(
                        rows.at[nb], acc.at[idxd.at[jm, q - 1]],
                        ssems[nb]).wait()
                if q < QB - 1:
                    pltpu.async_copy(h_ref.at[idxs.at[jm, q + 1]],
                                     rows.at[nb], gsems[nb])
                else:
                    @pl.when(jb + 1 < nblk)
                    def _():
                        pltpu.make_async_copy(
                            src_ref.at[pl.ds(row0, QB)], idxs.at[jn],
                            isem).wait()
                        pltpu.make_async_copy(
                            dst_ref.at[pl.ds(row0, QB)], idxd.at[jn],
                            isem).wait()
                        pltpu.async_copy(h_ref.at[idxs.at[jn, 0]],
                                         rows.at[nb], gsems[nb])
            return 0

        lax.fori_loop(0, nblk, body, 0)
        pltpu.make_async_copy(
            rows.at[1], acc.at[idxd.at[(nblk - 1) % 2, QB - 1]],
            ssems[1]).wait()
        plsc.subcore_barrier()
        pltpu.sync_copy(acc.at[pl.ds(s * RS, RS)],
                        out_ref.at[pl.ds(c * NPAD + s * RS, RS)])
        plsc.subcore_barrier()


def _sc_aggregate(hs, srcs, dsts, npws, hd):
    z = jnp.zeros((NPAD, hd), jnp.float32)
    out_t = [jax.ShapeDtypeStruct((2 * NPAD, hd), jnp.float32)] * 3
    fn = pl.kernel(
        functools.partial(_agg_body, tuple(npws), hd),
        out_type=out_t,
        mesh=_mesh(),
        scratch_types=[
            pltpu.VMEM_SHARED((NPAD, hd), jnp.float32),
            pltpu.VMEM((2, QB, CH), jnp.int32),
            pltpu.VMEM((2, QB, CH), jnp.int32),
            pltpu.VMEM((2, CH, hd), jnp.float32),
            pltpu.SemaphoreType.DMA,
            pltpu.SemaphoreType.DMA,
            pltpu.SemaphoreType.DMA,
            pltpu.SemaphoreType.DMA,
            pltpu.SemaphoreType.DMA,
        ],
        compiler_params=pltpu.CompilerParams(use_tc_tiling_on_sc=(hd == H)),
        name=f"sc_gcn_agg_{hd}",
    )
    return fn(*hs, *srcs, *dsts, z)


_BLK = 1000
_G = N // _BLK
_DOT = dict(preferred_element_type=jnp.float32,
            precision=jax.lax.Precision.HIGHEST)


def _tc1_body(x_ref, w_ref, degp_ref, hp_ref, dinv_ref):
    deg = degp_ref[0, :, 0:1] + degp_ref[1, :, 0:1] + 1.0
    dinv = lax.rsqrt(deg)
    h = lax.dot_general(x_ref[...], w_ref[...], (((1,), (0,)), ((), ())),
                        **_DOT)
    hp_ref[...] = h * dinv
    dinv_ref[...] = dinv


def _tc1(x, w1, degp):
    d = x.shape[1]
    return pl.pallas_call(
        _tc1_body,
        grid=(_G,),
        in_specs=[
            pl.BlockSpec((_BLK, d), lambda i: (i, 0)),
            pl.BlockSpec((d, H), lambda i: (0, 0)),
            pl.BlockSpec((2, _BLK, DW), lambda i: (0, i, 0)),
        ],
        out_specs=[
            pl.BlockSpec((_BLK, H), lambda i: (i, 0)),
            pl.BlockSpec((_BLK, 1), lambda i: (i, 0)),
        ],
        out_shape=[
            jax.ShapeDtypeStruct((N, H), jnp.float32),
            jax.ShapeDtypeStruct((N, 1), jnp.float32),
        ],
    )(x, w1, degp)


def _tc2_body(sp_ref, hp_ref, dinv_ref, b1_ref, w2_ref, out_ref):
    dinv = dinv_ref[...]
    y = (sp_ref[0] + sp_ref[1] + hp_ref[...]) * dinv + b1_ref[...]
    y = jnp.maximum(y, 0.0)
    h2 = lax.dot_general(y, w2_ref[...], (((1,), (0,)), ((), ())), **_DOT)
    out_ref[...] = h2 * dinv


def _tc2(sp, hp, dinv, b1, w2):
    return pl.pallas_call(
        _tc2_body,
        grid=(_G,),
        in_specs=[
            pl.BlockSpec((2, _BLK, H), lambda i: (0, i, 0)),
            pl.BlockSpec((_BLK, H), lambda i: (i, 0)),
            pl.BlockSpec((_BLK, 1), lambda i: (i, 0)),
            pl.BlockSpec((1, H), lambda i: (0, 0)),
            pl.BlockSpec((H, OUT), lambda i: (0, 0)),
        ],
        out_specs=pl.BlockSpec((_BLK, OUT), lambda i: (i, 0)),
        out_shape=jax.ShapeDtypeStruct((N, OUT), jnp.float32),
    )(sp, hp, dinv, b1, w2)


def _tc3_body(sp0, hp0, di0, bb0, bt0,
              sp1, hp1, di1, bb1, bt1,
              sp2, hp2, di2, bb2, bt2,
              demo_ref, f1w, f1b, f2w, f2b, f3w, f3b,
              out_ref, pooled, counts):
    i = pl.program_id(0)

    @pl.when(i == 0)
    def _():
        pooled[...] = jnp.zeros_like(pooled)
        counts[...] = jnp.zeros_like(counts)

    ones_col = jnp.ones((_BLK, 1), jnp.float32)
    for b, (sp, hp, di, bb, bt) in enumerate((
            (sp0, hp0, di0, bb0, bt0),
            (sp1, hp1, di1, bb1, bt1),
            (sp2, hp2, di2, bb2, bt2))):
        y = (sp[0] + sp[1] + hp[...]) * di[...] + bb[...]
        y = jnp.maximum(y, 0.0)
        gids = lax.broadcasted_iota(jnp.int32, (B, _BLK), 0)
        m = (gids == bt[0]).astype(jnp.float32)
        pooled[:, b * OUT:(b + 1) * OUT] += lax.dot_general(
            m, y, (((1,), (0,)), ((), ())), **_DOT)
        counts[:, b:b + 1] += lax.dot_general(
            m, ones_col, (((1,), (0,)), ((), ())), **_DOT)

    @pl.when(i == _G - 1)
    def _():
        cnt = jnp.maximum(counts[...], 1.0)
        h = f1b[...]
        for b in range(3):
            p = pooled[:, b * OUT:(b + 1) * OUT] / cnt[:, b:b + 1]
            h = h + lax.dot_general(
                p, f1w[b * OUT:(b + 1) * OUT, :],
                (((1,), (0,)), ((), ())), **_DOT)
        h = h + lax.dot_general(demo_ref[...], f1w[3 * OUT:, :],
                                (((1,), (0,)), ((), ())), **_DOT)
        h = jnp.maximum(h, 0.0)
        h = jnp.maximum(lax.dot_general(h, f2w[...],
                                        (((1,), (0,)), ((), ())), **_DOT)
                        + f2b[...], 0.0)
        out_ref[...] = lax.dot_general(h, f3w[...],
                                       (((1,), (0,)), ((), ())), **_DOT) \
            + f3b[...]


def _tc3(branches, demo, f1w, f1b, f2w, f2b, f3w, f3b):
    in_arrays = []
    in_specs = []
    for sp, hp, dinv, b2, bat3 in branches:
        in_arrays += [sp, hp, dinv, b2, bat3]
        in_specs += [
            pl.BlockSpec((2, _BLK, OUT), lambda i: (0, i, 0)),
            pl.BlockSpec((_BLK, OUT), lambda i: (i, 0)),
            pl.BlockSpec((_BLK, 1), lambda i: (i, 0)),
            pl.BlockSpec((1, OUT), lambda i: (0, 0)),
            pl.BlockSpec((1, 1, _BLK), lambda i: (i, 0, 0)),
        ]
    in_arrays += [demo, f1w, f1b, f2w, f2b, f3w, f3b]
    in_specs += [
        pl.BlockSpec((B, 16), lambda i: (0, 0)),
        pl.BlockSpec((3 * OUT + 16, B), lambda i: (0, 0)),
        pl.BlockSpec((1, B), lambda i: (0, 0)),
        pl.BlockSpec((B, 32), lambda i: (0, 0)),
        pl.BlockSpec((1, 32), lambda i: (0, 0)),
        pl.BlockSpec((32, 2), lambda i: (0, 0)),
        pl.BlockSpec((1, 2), lambda i: (0, 0)),
    ]
    return pl.pallas_call(
        _tc3_body,
        grid=(_G,),
        in_specs=in_specs,
        out_specs=pl.BlockSpec((B, 2), lambda i: (0, 0)),
        out_shape=jax.ShapeDtypeStruct((B, 2), jnp.float32),
        scratch_shapes=[
            pltpu.VMEM((B, 3 * OUT), jnp.float32),
            pltpu.VMEM((B, 8), jnp.float32),
        ],
    )(*in_arrays)


def kernel(x_desikan, edge_index_desikan, batch_desikan,
           x_destrieux, edge_index_destrieux, batch_destrieux,
           x_fuzzy, edge_index_fuzzy, batch_fuzzy,
           demographic,
           W1_des, b1_des, W2_des, b2_des,
           W1_det, b1_det, W2_det, b2_det,
           W1_fuz, b1_fuz, W2_fuz, b2_fuz,
           fc1_W, fc1_b, fc2_W, fc2_b, fc3_W, fc3_b):
    xs = (x_desikan, x_destrieux, x_fuzzy)
    eis = (edge_index_desikan, edge_index_destrieux, edge_index_fuzzy)
    bats = (batch_desikan, batch_destrieux, batch_fuzzy)
    w1s, b1s = (W1_des, W1_det, W1_fuz), (b1_des, b1_det, b1_fuz)
    w2s, b2s = (W2_des, W2_det, W2_fuz), (b2_des, b2_det, b2_fuz)

    srcs, dsts, npws = [], [], []
    for ei in eis:
        e_pad = -(-ei.shape[1] // E_ALIGN) * E_ALIGN
        s, d = _pad_edges(ei, e_pad)
        srcs.append(s)
        dsts.append(d)
        npws.append(e_pad // (NW * CH))
    srcs, dsts, npws = tuple(srcs), tuple(dsts), tuple(npws)

    degps = _sc_degrees(dsts, npws)
    degps = [p.reshape(2, NPAD, DW) for p in degps]

    h1ps, dinvs = [], []
    for x, w1, degp in zip(xs, w1s, degps):
        hp, dinv = _tc1(x, w1, degp)
        h1ps.append(hp)
        dinvs.append(dinv)

    s1ps = _sc_aggregate(tuple(h1ps), srcs, dsts, npws, H)
    s1ps = [p.reshape(2, NPAD, H) for p in s1ps]

    h2ps = []
    for sp, hp, dinv, b1, w2 in zip(s1ps, h1ps, dinvs, b1s, w2s):
        h2ps.append(_tc2(sp, hp, dinv, b1.reshape(1, H), w2))

    s2ps = _sc_aggregate(tuple(h2ps), srcs, dsts, npws, OUT)
    s2ps = [p.reshape(2, NPAD, OUT) for p in s2ps]

    branches = []
    for sp, hp, dinv, b2, bat in zip(s2ps, h2ps, dinvs, b2s, bats):
        branches.append((sp, hp, dinv, b2.reshape(1, OUT),
                         bat.reshape(_G, 1, _BLK)))

    return _tc3(branches, demographic, fc1_W, fc1_b.reshape(1, B),
                fc2_W, fc2_b.reshape(1, 32), fc3_W, fc3_b.reshape(1, 2))


# speedup vs baseline: 27.0864x; 1.0729x over previous
import functools

import jax
import jax.numpy as jnp
import numpy as np
from jax import lax
from jax.experimental import pallas as pl
from jax.experimental.pallas import tpu as pltpu
from jax.experimental.pallas import tpu_sc as plsc

N = 10000
B = 64
H = 128
OUT = 64
NC = 2
NS = 16
NW = NC * NS
CH = 128
DUM = 512
NPAD = 10752
RS = NPAD // NS
QB = 8
E_ALIGN = NW * CH * QB
DW = 8

@functools.cache
def _mesh():
    return plsc.VectorSubcoreMesh(core_axis_name="c", subcore_axis_name="s",
                                  num_cores=NC, num_subcores=NS)


def _pad_edges(ei, e_pad):
    e = ei.shape[1]
    k = jnp.arange(e_pad - e, dtype=jnp.int32)
    src = jnp.concatenate([ei[0], k % np.int32(N)]).reshape(e_pad // CH, CH)
    dst = jnp.concatenate([ei[1], np.int32(N) + (k % np.int32(DUM))])
    return src, dst.reshape(e_pad // CH, CH)


def _deg_body(npws, d0, d1, d2, z_ref, ones_ref, o0, o1, o2,
              acc, ones_v, idxd, isem, ssem):
    c = lax.axis_index("c")
    s = lax.axis_index("s")
    w = s * NC + c
    pltpu.sync_copy(ones_ref, ones_v)
    for dst_ref, out_ref, npw in zip((d0, d1, d2), (o0, o1, o2), npws):
        nblk = npw // QB
        pltpu.sync_copy(z_ref.at[pl.ds(s * RS, RS)], acc.at[pl.ds(s * RS, RS)])
        plsc.subcore_barrier()
        row0 = w * npw
        pltpu.sync_copy(dst_ref.at[pl.ds(row0, QB)], idxd.at[0])

        def body(jb, _):
            jm = jb % 2
            jn = (jb + 1) % 2

            @pl.when(jb + 1 < nblk)
            def _():
                pltpu.async_copy(
                    dst_ref.at[pl.ds(row0 + (jb + 1) * QB, QB)],
                    idxd.at[jn], isem)

            for q in range(QB):
                pltpu.async_copy(ones_v, acc.at[idxd.at[jm, q]], ssem,
                                 add=True)
            for q in range(QB):
                pltpu.make_async_copy(ones_v, acc.at[idxd.at[jm, q]],
                                      ssem).wait()

            @pl.when(jb + 1 < nblk)
            def _():
                pltpu.make_async_copy(
                    dst_ref.at[pl.ds(row0, QB)], idxd.at[jn], isem).wait()
            return 0

        lax.fori_loop(0, nblk, body, 0)
        plsc.subcore_barrier()
        pltpu.sync_copy(acc.at[pl.ds(s * RS, RS)],
                        out_ref.at[pl.ds(c * NPAD + s * RS, RS)])
        plsc.subcore_barrier()


def _sc_degrees(dsts, npws):
    z = jnp.zeros((NPAD, DW), jnp.float32)
    ones = jnp.ones((CH, DW), jnp.float32)
    out_t = [jax.ShapeDtypeStruct((2 * NPAD, DW), jnp.float32)] * 3
    fn = pl.kernel(
        functools.partial(_deg_body, tuple(npws)),
        out_type=out_t,
        mesh=_mesh(),
        scratch_types=[
            pltpu.VMEM_SHARED((NPAD, DW), jnp.float32),
            pltpu.VMEM((CH, DW), jnp.float32),
            pltpu.VMEM((2, QB, CH), jnp.int32),
            pltpu.SemaphoreType.DMA,
            pltpu.SemaphoreType.DMA,
        ],
        compiler_params=pltpu.CompilerParams(use_tc_tiling_on_sc=False),
        name="sc_degrees",
    )
    return fn(*dsts, z, ones)


def _agg_body(npws, hd, h0, h1, h2, s0, s1, s2, d0, d1, d2, z_ref,
              o0, o1, o2, acc, idxs, idxd, rows, gs0, gs1, ss0, ss1, isem):
    c = lax.axis_index("c")
    s = lax.axis_index("s")
    w = s * NC + c
    gsems = (gs0, gs1)
    ssems = (ss0, ss1)
    for h_ref, src_ref, dst_ref, out_ref, npw in zip(
            (h0, h1, h2), (s0, s1, s2), (d0, d1, d2), (o0, o1, o2), npws):
        nblk = npw // QB
        pltpu.sync_copy(z_ref.at[pl.ds(s * RS, RS)], acc.at[pl.ds(s * RS, RS)])
        plsc.subcore_barrier()

        row0 = w * npw
        pltpu.sync_copy(src_ref.at[pl.ds(row0, QB)], idxs.at[0])
        pltpu.sync_copy(dst_ref.at[pl.ds(row0, QB)], idxd.at[0])
        pltpu.async_copy(h_ref.at[idxs.at[0, 0]], rows.at[0], gsems[0])

        def body(jb, _):
            jm = jb % 2
            jn = (jb + 1) % 2

            @pl.when(jb > 0)
            def _():
                pltpu.make_async_copy(
                    rows.at[1], acc.at[idxd.at[jn, QB - 1]], ssems[1]).wait()

            @pl.when(jb + 1 < nblk)
            def _():
                pltpu.async_copy(
                    src_ref.at[pl.ds(row0 + (jb + 1) * QB, QB)],
                    idxs.at[jn], isem)
                pltpu.async_copy(
                    dst_ref.at[pl.ds(row0 + (jb + 1) * QB, QB)],
                    idxd.at[jn], isem)

            for q in range(QB):
                b = q % 2
                nb = 1 - b
                pltpu.make_async_copy(
                    h_ref.at[idxs.at[jm, q]], rows.at[b], gsems[b]).wait()
                pltpu.async_copy(rows.at[b], acc.at[idxd.at[jm, q]],
                                 ssems[b], add=True)
                if 0 < q:
                    pltpu.make_async_copy(
                        rows.at[nb], acc.at[idxd.at[jm, q - 1]],
                        ssems[nb]).wait()
                if q < QB - 1:
                    pltpu.async_copy(h_ref.at[idxs.at[jm, q + 1]],
                                     rows.at[nb], gsems[nb])
                else:
                    @pl.when(jb + 1 < nblk)
                    def _():
                        pltpu.make_async_copy(
                            src_ref.at[pl.ds(row0, QB)], idxs.at[jn],
                            isem).wait()
                        pltpu.make_async_copy(
                            dst_ref.at[pl.ds(row0, QB)], idxd.at[jn],
                            isem).wait()
                        pltpu.async_copy(h_ref.at[idxs.at[jn, 0]],
                                         rows.at[nb], gsems[nb])
            return 0

        lax.fori_loop(0, nblk, body, 0)
        pltpu.make_async_copy(
            rows.at[1], acc.at[idxd.at[(nblk - 1) % 2, QB - 1]],
            ssems[1]).wait()
        plsc.subcore_barrier()
        pltpu.sync_copy(acc.at[pl.ds(s * RS, RS)],
                        out_ref.at[pl.ds(c * NPAD + s * RS, RS)])
        plsc.subcore_barrier()


def _sc_aggregate(hs, srcs, dsts, npws, hd):
    z = jnp.zeros((NPAD, hd), jnp.float32)
    out_t = [jax.ShapeDtypeStruct((2 * NPAD, hd), jnp.float32)] * 3
    fn = pl.kernel(
        functools.partial(_agg_body, tuple(npws), hd),
        out_type=out_t,
        mesh=_mesh(),
        scratch_types=[
            pltpu.VMEM_SHARED((NPAD, hd), jnp.float32),
            pltpu.VMEM((2, QB, CH), jnp.int32),
            pltpu.VMEM((2, QB, CH), jnp.int32),
            pltpu.VMEM((2, CH, hd), jnp.float32),
            pltpu.SemaphoreType.DMA,
            pltpu.SemaphoreType.DMA,
            pltpu.SemaphoreType.DMA,
            pltpu.SemaphoreType.DMA,
            pltpu.SemaphoreType.DMA,
        ],
        compiler_params=pltpu.CompilerParams(use_tc_tiling_on_sc=(hd == H)),
        name=f"sc_gcn_agg_{hd}",
    )
    return fn(*hs, *srcs, *dsts, z)


_BLK = 1000
_G = N // _BLK
_DOT = dict(preferred_element_type=jnp.float32)


def _tc1_body(*refs):
    for b in range(3):
        x_ref, w_ref, degp_ref = refs[3 * b:3 * b + 3]
        hp_ref, dinv_ref = refs[9 + 2 * b:9 + 2 * b + 2]
        deg = degp_ref[0, :, 0:1] + degp_ref[1, :, 0:1] + 1.0
        dinv = lax.rsqrt(deg)
        h = lax.dot_general(x_ref[...], w_ref[...], (((1,), (0,)), ((), ())),
                            **_DOT)
        hp_ref[...] = h * dinv
        dinv_ref[...] = dinv


def _tc1(xs, w1s, degps):
    in_arrays, in_specs = [], []
    for x, w1, degp in zip(xs, w1s, degps):
        d = x.shape[1]
        in_arrays += [x, w1, degp]
        in_specs += [
            pl.BlockSpec((_BLK, d), lambda i: (i, 0)),
            pl.BlockSpec((d, H), lambda i: (0, 0)),
            pl.BlockSpec((2, _BLK, DW), lambda i: (0, i, 0)),
        ]
    return pl.pallas_call(
        _tc1_body,
        grid=(_G,),
        in_specs=in_specs,
        out_specs=[
            pl.BlockSpec((_BLK, H), lambda i: (i, 0)),
            pl.BlockSpec((_BLK, 1), lambda i: (i, 0)),
        ] * 3,
        out_shape=[
            jax.ShapeDtypeStruct((N, H), jnp.float32),
            jax.ShapeDtypeStruct((N, 1), jnp.float32),
        ] * 3,
    )(*in_arrays)


def _tc2_body(*refs):
    for b in range(3):
        sp_ref, hp_ref, dinv_ref, b1_ref, w2_ref = refs[5 * b:5 * b + 5]
        out_ref = refs[15 + b]
        dinv = dinv_ref[...]
        y = (sp_ref[0] + sp_ref[1] + hp_ref[...]) * dinv + b1_ref[...]
        y = jnp.maximum(y, 0.0)
        h2 = lax.dot_general(y, w2_ref[...], (((1,), (0,)), ((), ())), **_DOT)
        out_ref[...] = h2 * dinv


def _tc2(sps, hps, dinvs, b1s, w2s):
    in_arrays, in_specs = [], []
    for sp, hp, dinv, b1, w2 in zip(sps, hps, dinvs, b1s, w2s):
        in_arrays += [sp, hp, dinv, b1.reshape(1, H), w2]
        in_specs += [
            pl.BlockSpec((2, _BLK, H), lambda i: (0, i, 0)),
            pl.BlockSpec((_BLK, H), lambda i: (i, 0)),
            pl.BlockSpec((_BLK, 1), lambda i: (i, 0)),
            pl.BlockSpec((1, H), lambda i: (0, 0)),
            pl.BlockSpec((H, OUT), lambda i: (0, 0)),
        ]
    return pl.pallas_call(
        _tc2_body,
        grid=(_G,),
        in_specs=in_specs,
        out_specs=[pl.BlockSpec((_BLK, OUT), lambda i: (i, 0))] * 3,
        out_shape=[jax.ShapeDtypeStruct((N, OUT), jnp.float32)] * 3,
    )(*in_arrays)


def _tc3_body(sp0, hp0, di0, bb0, bt0,
              sp1, hp1, di1, bb1, bt1,
              sp2, hp2, di2, bb2, bt2,
              demo_ref, f1w, f1b, f2w, f2b, f3w, f3b,
              out_ref, pooled, counts):
    i = pl.program_id(0)

    @pl.when(i == 0)
    def _():
        pooled[...] = jnp.zeros_like(pooled)
        counts[...] = jnp.zeros_like(counts)

    ones_col = jnp.ones((_BLK, 1), jnp.float32)
    for b, (sp, hp, di, bb, bt) in enumerate((
            (sp0, hp0, di0, bb0, bt0),
            (sp1, hp1, di1, bb1, bt1),
            (sp2, hp2, di2, bb2, bt2))):
        y = (sp[0] + sp[1] + hp[...]) * di[...] + bb[...]
        y = jnp.maximum(y, 0.0)
        gids = lax.broadcasted_iota(jnp.int32, (B, _BLK), 0)
        m = (gids == bt[0]).astype(jnp.float32)
        pooled[:, b * OUT:(b + 1) * OUT] += lax.dot_general(
            m, y, (((1,), (0,)), ((), ())), **_DOT)
        counts[:, b:b + 1] += lax.dot_general(
            m, ones_col, (((1,), (0,)), ((), ())), **_DOT)

    @pl.when(i == _G - 1)
    def _():
        cnt = jnp.maximum(counts[...], 1.0)
        h = f1b[...]
        for b in range(3):
            p = pooled[:, b * OUT:(b + 1) * OUT] / cnt[:, b:b + 1]
            h = h + lax.dot_general(
                p, f1w[b * OUT:(b + 1) * OUT, :],
                (((1,), (0,)), ((), ())), **_DOT)
        h = h + lax.dot_general(demo_ref[...], f1w[3 * OUT:, :],
                                (((1,), (0,)), ((), ())), **_DOT)
        h = jnp.maximum(h, 0.0)
        h = jnp.maximum(lax.dot_general(h, f2w[...],
                                        (((1,), (0,)), ((), ())), **_DOT)
                        + f2b[...], 0.0)
        out_ref[...] = lax.dot_general(h, f3w[...],
                                       (((1,), (0,)), ((), ())), **_DOT) \
            + f3b[...]


def _tc3(branches, demo, f1w, f1b, f2w, f2b, f3w, f3b):
    in_arrays = []
    in_specs = []
    for sp, hp, dinv, b2, bat3 in branches:
        in_arrays += [sp, hp, dinv, b2, bat3]
        in_specs += [
            pl.BlockSpec((2, _BLK, OUT), lambda i: (0, i, 0)),
            pl.BlockSpec((_BLK, OUT), lambda i: (i, 0)),
            pl.BlockSpec((_BLK, 1), lambda i: (i, 0)),
            pl.BlockSpec((1, OUT), lambda i: (0, 0)),
            pl.BlockSpec((1, 1, _BLK), lambda i: (i, 0, 0)),
        ]
    in_arrays += [demo, f1w, f1b, f2w, f2b, f3w, f3b]
    in_specs += [
        pl.BlockSpec((B, 16), lambda i: (0, 0)),
        pl.BlockSpec((3 * OUT + 16, B), lambda i: (0, 0)),
        pl.BlockSpec((1, B), lambda i: (0, 0)),
        pl.BlockSpec((B, 32), lambda i: (0, 0)),
        pl.BlockSpec((1, 32), lambda i: (0, 0)),
        pl.BlockSpec((32, 2), lambda i: (0, 0)),
        pl.BlockSpec((1, 2), lambda i: (0, 0)),
    ]
    return pl.pallas_call(
        _tc3_body,
        grid=(_G,),
        in_specs=in_specs,
        out_specs=pl.BlockSpec((B, 2), lambda i: (0, 0)),
        out_shape=jax.ShapeDtypeStruct((B, 2), jnp.float32),
        scratch_shapes=[
            pltpu.VMEM((B, 3 * OUT), jnp.float32),
            pltpu.VMEM((B, 8), jnp.float32),
        ],
    )(*in_arrays)


def kernel(x_desikan, edge_index_desikan, batch_desikan,
           x_destrieux, edge_index_destrieux, batch_destrieux,
           x_fuzzy, edge_index_fuzzy, batch_fuzzy,
           demographic,
           W1_des, b1_des, W2_des, b2_des,
           W1_det, b1_det, W2_det, b2_det,
           W1_fuz, b1_fuz, W2_fuz, b2_fuz,
           fc1_W, fc1_b, fc2_W, fc2_b, fc3_W, fc3_b):
    xs = (x_desikan, x_destrieux, x_fuzzy)
    eis = (edge_index_desikan, edge_index_destrieux, edge_index_fuzzy)
    bats = (batch_desikan, batch_destrieux, batch_fuzzy)
    w1s, b1s = (W1_des, W1_det, W1_fuz), (b1_des, b1_det, b1_fuz)
    w2s, b2s = (W2_des, W2_det, W2_fuz), (b2_des, b2_det, b2_fuz)

    srcs, dsts, npws = [], [], []
    for ei in eis:
        e_pad = -(-ei.shape[1] // E_ALIGN) * E_ALIGN
        s, d = _pad_edges(ei, e_pad)
        srcs.append(s)
        dsts.append(d)
        npws.append(e_pad // (NW * CH))
    srcs, dsts, npws = tuple(srcs), tuple(dsts), tuple(npws)

    degps = _sc_degrees(dsts, npws)
    degps = [p.reshape(2, NPAD, DW) for p in degps]

    tc1_out = _tc1(xs, w1s, degps)
    h1ps = [tc1_out[0], tc1_out[2], tc1_out[4]]
    dinvs = [tc1_out[1], tc1_out[3], tc1_out[5]]

    s1ps = _sc_aggregate(tuple(h1ps), srcs, dsts, npws, H)
    s1ps = [p.reshape(2, NPAD, H) for p in s1ps]

    h2ps = _tc2(s1ps, h1ps, dinvs, b1s, w2s)

    s2ps = _sc_aggregate(tuple(h2ps), srcs, dsts, npws, OUT)
    s2ps = [p.reshape(2, NPAD, OUT) for p in s2ps]

    branches = []
    for sp, hp, dinv, b2, bat in zip(s2ps, h2ps, dinvs, b2s, bats):
        branches.append((sp, hp, dinv, b2.reshape(1, OUT),
                         bat.reshape(_G, 1, _BLK)))

    return _tc3(branches, demographic, fc1_W, fc1_b.reshape(1, B),
                fc2_W, fc2_b.reshape(1, 32), fc3_W, fc3_b.reshape(1, 2))


# speedup vs baseline: 28.1347x; 1.0387x over previous
import functools

import jax
import jax.numpy as jnp
import numpy as np
from jax import lax
from jax.experimental import pallas as pl
from jax.experimental.pallas import tpu as pltpu
from jax.experimental.pallas import tpu_sc as plsc

N = 10000
B = 64
H = 128
OUT = 64
NC = 2
NS = 16
NW = NC * NS
CH = 128
DUM = 512
NPAD = 10752
RS = NPAD // NS
QB = 8
E_ALIGN = NW * CH * QB
DW = 8

@functools.cache
def _mesh():
    return plsc.VectorSubcoreMesh(core_axis_name="c", subcore_axis_name="s",
                                  num_cores=NC, num_subcores=NS)


def _pad_edges(ei, e_pad):
    e = ei.shape[1]
    k = jnp.arange(e_pad - e, dtype=jnp.int32)
    src = jnp.concatenate([ei[0], k % np.int32(N)]).reshape(e_pad // CH, CH)
    dst = jnp.concatenate([ei[1], np.int32(N) + (k % np.int32(DUM))])
    return src, dst.reshape(e_pad // CH, CH)


def _deg_body(npws, d0, d1, d2, z_ref, ones_ref, o0, o1, o2,
              acc, ones_v, idxd, isem, ssem):
    c = lax.axis_index("c")
    s = lax.axis_index("s")
    w = s * NC + c
    pltpu.sync_copy(ones_ref, ones_v)
    for dst_ref, out_ref, npw in zip((d0, d1, d2), (o0, o1, o2), npws):
        nblk = npw // QB
        pltpu.sync_copy(z_ref.at[pl.ds(s * RS, RS)], acc.at[pl.ds(s * RS, RS)])
        plsc.subcore_barrier()
        row0 = w * npw
        pltpu.sync_copy(dst_ref.at[pl.ds(row0, QB)], idxd.at[0])

        def body(jb, _):
            jm = jb % 2
            jn = (jb + 1) % 2

            @pl.when(jb + 1 < nblk)
            def _():
                pltpu.async_copy(
                    dst_ref.at[pl.ds(row0 + (jb + 1) * QB, QB)],
                    idxd.at[jn], isem)

            for q in range(QB):
                pltpu.async_copy(ones_v, acc.at[idxd.at[jm, q]], ssem,
                                 add=True)
            for q in range(QB):
                pltpu.make_async_copy(ones_v, acc.at[idxd.at[jm, q]],
                                      ssem).wait()

            @pl.when(jb + 1 < nblk)
            def _():
                pltpu.make_async_copy(
                    dst_ref.at[pl.ds(row0, QB)], idxd.at[jn], isem).wait()
            return 0

        lax.fori_loop(0, nblk, body, 0)
        plsc.subcore_barrier()
        pltpu.sync_copy(acc.at[pl.ds(s * RS, RS)],
                        out_ref.at[pl.ds(c * NPAD + s * RS, RS)])
        plsc.subcore_barrier()


def _sc_degrees(dsts, npws):
    z = jnp.zeros((NPAD, DW), jnp.float32)
    ones = jnp.ones((CH, DW), jnp.float32)
    out_t = [jax.ShapeDtypeStruct((2 * NPAD, DW), jnp.float32)] * 3
    fn = pl.kernel(
        functools.partial(_deg_body, tuple(npws)),
        out_type=out_t,
        mesh=_mesh(),
        scratch_types=[
            pltpu.VMEM_SHARED((NPAD, DW), jnp.float32),
            pltpu.VMEM((CH, DW), jnp.float32),
            pltpu.VMEM((2, QB, CH), jnp.int32),
            pltpu.SemaphoreType.DMA,
            pltpu.SemaphoreType.DMA,
        ],
        compiler_params=pltpu.CompilerParams(use_tc_tiling_on_sc=False),
        name="sc_degrees",
    )
    return fn(*dsts, z, ones)


def _agg_body(npws, hd, stage, *refs):
    (h0, h1, h2, s0, s1, s2, d0, d1, d2, z_ref, o0, o1, o2,
     acc, idxs, idxd, rows, gs0, gs1, ss0, ss1, isem) = refs[:22]
    tbl = refs[22] if stage else None
    c = lax.axis_index("c")
    s = lax.axis_index("s")
    w = s * NC + c
    gsems = (gs0, gs1)
    ssems = (ss0, ss1)
    for h_hbm, src_ref, dst_ref, out_ref, npw in zip(
            (h0, h1, h2), (s0, s1, s2), (d0, d1, d2), (o0, o1, o2), npws):
        nblk = npw // QB
        pltpu.sync_copy(z_ref.at[pl.ds(s * RS, RS)], acc.at[pl.ds(s * RS, RS)])
        if stage:
            pltpu.sync_copy(h_hbm.at[pl.ds(s * (N // NS), N // NS)],
                            tbl.at[pl.ds(s * (N // NS), N // NS)])
            h_ref = tbl
        else:
            h_ref = h_hbm
        plsc.subcore_barrier()

        row0 = w * npw
        pltpu.sync_copy(src_ref.at[pl.ds(row0, QB)], idxs.at[0])
        pltpu.sync_copy(dst_ref.at[pl.ds(row0, QB)], idxd.at[0])
        pltpu.async_copy(h_ref.at[idxs.at[0, 0]], rows.at[0], gsems[0])

        def body(jb, _):
            jm = jb % 2
            jn = (jb + 1) % 2

            @pl.when(jb > 0)
            def _():
                pltpu.make_async_copy(
                    rows.at[1], acc.at[idxd.at[jn, QB - 1]], ssems[1]).wait()

            @pl.when(jb + 1 < nblk)
            def _():
                pltpu.async_copy(
                    src_ref.at[pl.ds(row0 + (jb + 1) * QB, QB)],
                    idxs.at[jn], isem)
                pltpu.async_copy(
                    dst_ref.at[pl.ds(row0 + (jb + 1) * QB, QB)],
                    idxd.at[jn], isem)

            for q in range(QB):
                b = q % 2
                nb = 1 - b
                pltpu.make_async_copy(
                    h_ref.at[idxs.at[jm, q]], rows.at[b], gsems[b]).wait()
                pltpu.async_copy(rows.at[b], acc.at[idxd.at[jm, q]],
                                 ssems[b], add=True)
                if 0 < q:
                    pltpu.make_async_copy(
                        rows.at[nb], acc.at[idxd.at[jm, q - 1]],
                        ssems[nb]).wait()
                if q < QB - 1:
                    pltpu.async_copy(h_ref.at[idxs.at[jm, q + 1]],
                                     rows.at[nb], gsems[nb])
                else:
                    @pl.when(jb + 1 < nblk)
                    def _():
                        pltpu.make_async_copy(
                            src_ref.at[pl.ds(row0, QB)], idxs.at[jn],
                            isem).wait()
                        pltpu.make_async_copy(
                            dst_ref.at[pl.ds(row0, QB)], idxd.at[jn],
                            isem).wait()
                        pltpu.async_copy(h_ref.at[idxs.at[jn, 0]],
                                         rows.at[nb], gsems[nb])
            return 0

        lax.fori_loop(0, nblk, body, 0)
        pltpu.make_async_copy(
            rows.at[1], acc.at[idxd.at[(nblk - 1) % 2, QB - 1]],
            ssems[1]).wait()
        plsc.subcore_barrier()
        pltpu.sync_copy(acc.at[pl.ds(s * RS, RS)],
                        out_ref.at[pl.ds(c * NPAD + s * RS, RS)])
        plsc.subcore_barrier()


def _sc_aggregate(hs, srcs, dsts, npws, hd):
    z = jnp.zeros((NPAD, hd), jnp.float32)
    stage = hd * (N + NPAD) * 4 <= 6 * 2**20
    out_t = [jax.ShapeDtypeStruct((2 * NPAD, hd), jnp.float32)] * 3
    scratch = [
        pltpu.VMEM_SHARED((NPAD, hd), jnp.float32),
        pltpu.VMEM((2, QB, CH), jnp.int32),
        pltpu.VMEM((2, QB, CH), jnp.int32),
        pltpu.VMEM((2, CH, hd), jnp.float32),
        pltpu.SemaphoreType.DMA,
        pltpu.SemaphoreType.DMA,
        pltpu.SemaphoreType.DMA,
        pltpu.SemaphoreType.DMA,
        pltpu.SemaphoreType.DMA,
    ]
    if stage:
        scratch.append(pltpu.VMEM_SHARED((N, hd), jnp.float32))
    fn = pl.kernel(
        functools.partial(_agg_body, tuple(npws), hd, stage),
        out_type=out_t,
        mesh=_mesh(),
        scratch_types=scratch,
        compiler_params=pltpu.CompilerParams(use_tc_tiling_on_sc=(hd == H)),
        name=f"sc_gcn_agg_{hd}",
    )
    return fn(*hs, *srcs, *dsts, z)


_BLK = 1000
_G = N // _BLK
_DOT = dict(preferred_element_type=jnp.float32)


def _tc1_body(*refs):
    for b in range(3):
        x_ref, w_ref, degp_ref = refs[3 * b:3 * b + 3]
        hp_ref, dinv_ref = refs[9 + 2 * b:9 + 2 * b + 2]
        deg = degp_ref[0, :, 0:1] + degp_ref[1, :, 0:1] + 1.0
        dinv = lax.rsqrt(deg)
        h = lax.dot_general(x_ref[...], w_ref[...], (((1,), (0,)), ((), ())),
                            **_DOT)
        hp_ref[...] = h * dinv
        dinv_ref[...] = dinv


def _tc1(xs, w1s, degps):
    in_arrays, in_specs = [], []
    for x, w1, degp in zip(xs, w1s, degps):
        d = x.shape[1]
        in_arrays += [x, w1, degp]
        in_specs += [
            pl.BlockSpec((_BLK, d), lambda i: (i, 0)),
            pl.BlockSpec((d, H), lambda i: (0, 0)),
            pl.BlockSpec((2, _BLK, DW), lambda i: (0, i, 0)),
        ]
    return pl.pallas_call(
        _tc1_body,
        grid=(_G,),
        in_specs=in_specs,
        out_specs=[
            pl.BlockSpec((_BLK, H), lambda i: (i, 0)),
            pl.BlockSpec((_BLK, 1), lambda i: (i, 0)),
        ] * 3,
        out_shape=[
            jax.ShapeDtypeStruct((N, H), jnp.float32),
            jax.ShapeDtypeStruct((N, 1), jnp.float32),
        ] * 3,
    )(*in_arrays)


def _tc2_body(*refs):
    for b in range(3):
        sp_ref, hp_ref, dinv_ref, b1_ref, w2_ref = refs[5 * b:5 * b + 5]
        out_ref = refs[15 + b]
        dinv = dinv_ref[...]
        y = (sp_ref[0] + sp_ref[1] + hp_ref[...]) * dinv + b1_ref[...]
        y = jnp.maximum(y, 0.0)
        h2 = lax.dot_general(y, w2_ref[...], (((1,), (0,)), ((), ())), **_DOT)
        out_ref[...] = h2 * dinv


def _tc2(sps, hps, dinvs, b1s, w2s):
    in_arrays, in_specs = [], []
    for sp, hp, dinv, b1, w2 in zip(sps, hps, dinvs, b1s, w2s):
        in_arrays += [sp, hp, dinv, b1.reshape(1, H), w2]
        in_specs += [
            pl.BlockSpec((2, _BLK, H), lambda i: (0, i, 0)),
            pl.BlockSpec((_BLK, H), lambda i: (i, 0)),
            pl.BlockSpec((_BLK, 1), lambda i: (i, 0)),
            pl.BlockSpec((1, H), lambda i: (0, 0)),
            pl.BlockSpec((H, OUT), lambda i: (0, 0)),
        ]
    return pl.pallas_call(
        _tc2_body,
        grid=(_G,),
        in_specs=in_specs,
        out_specs=[pl.BlockSpec((_BLK, OUT), lambda i: (i, 0))] * 3,
        out_shape=[jax.ShapeDtypeStruct((N, OUT), jnp.float32)] * 3,
    )(*in_arrays)


def _tc3_body(sp0, hp0, di0, bb0, bt0,
              sp1, hp1, di1, bb1, bt1,
              sp2, hp2, di2, bb2, bt2,
              demo_ref, f1w, f1b, f2w, f2b, f3w, f3b,
              out_ref, pooled, counts):
    i = pl.program_id(0)

    @pl.when(i == 0)
    def _():
        pooled[...] = jnp.zeros_like(pooled)
        counts[...] = jnp.zeros_like(counts)

    ones_col = jnp.ones((_BLK, 1), jnp.float32)
    for b, (sp, hp, di, bb, bt) in enumerate((
            (sp0, hp0, di0, bb0, bt0),
            (sp1, hp1, di1, bb1, bt1),
            (sp2, hp2, di2, bb2, bt2))):
        y = (sp[0] + sp[1] + hp[...]) * di[...] + bb[...]
        y = jnp.maximum(y, 0.0)
        gids = lax.broadcasted_iota(jnp.int32, (B, _BLK), 0)
        m = (gids == bt[0]).astype(jnp.float32)
        pooled[:, b * OUT:(b + 1) * OUT] += lax.dot_general(
            m, y, (((1,), (0,)), ((), ())), **_DOT)
        counts[:, b:b + 1] += lax.dot_general(
            m, ones_col, (((1,), (0,)), ((), ())), **_DOT)

    @pl.when(i == _G - 1)
    def _():
        cnt = jnp.maximum(counts[...], 1.0)
        h = f1b[...]
        for b in range(3):
            p = pooled[:, b * OUT:(b + 1) * OUT] / cnt[:, b:b + 1]
            h = h + lax.dot_general(
                p, f1w[b * OUT:(b + 1) * OUT, :],
                (((1,), (0,)), ((), ())), **_DOT)
        h = h + lax.dot_general(demo_ref[...], f1w[3 * OUT:, :],
                                (((1,), (0,)), ((), ())), **_DOT)
        h = jnp.maximum(h, 0.0)
        h = jnp.maximum(lax.dot_general(h, f2w[...],
                                        (((1,), (0,)), ((), ())), **_DOT)
                        + f2b[...], 0.0)
        out_ref[...] = lax.dot_general(h, f3w[...],
                                       (((1,), (0,)), ((), ())), **_DOT) \
            + f3b[...]


def _tc3(branches, demo, f1w, f1b, f2w, f2b, f3w, f3b):
    in_arrays = []
    in_specs = []
    for sp, hp, dinv, b2, bat3 in branches:
        in_arrays += [sp, hp, dinv, b2, bat3]
        in_specs += [
            pl.BlockSpec((2, _BLK, OUT), lambda i: (0, i, 0)),
            pl.BlockSpec((_BLK, OUT), lambda i: (i, 0)),
            pl.BlockSpec((_BLK, 1), lambda i: (i, 0)),
            pl.BlockSpec((1, OUT), lambda i: (0, 0)),
            pl.BlockSpec((1, 1, _BLK), lambda i: (i, 0, 0)),
        ]
    in_arrays += [demo, f1w, f1b, f2w, f2b, f3w, f3b]
    in_specs += [
        pl.BlockSpec((B, 16), lambda i: (0, 0)),
        pl.BlockSpec((3 * OUT + 16, B), lambda i: (0, 0)),
        pl.BlockSpec((1, B), lambda i: (0, 0)),
        pl.BlockSpec((B, 32), lambda i: (0, 0)),
        pl.BlockSpec((1, 32), lambda i: (0, 0)),
        pl.BlockSpec((32, 2), lambda i: (0, 0)),
        pl.BlockSpec((1, 2), lambda i: (0, 0)),
    ]
    return pl.pallas_call(
        _tc3_body,
        grid=(_G,),
        in_specs=in_specs,
        out_specs=pl.BlockSpec((B, 2), lambda i: (0, 0)),
        out_shape=jax.ShapeDtypeStruct((B, 2), jnp.float32),
        scratch_shapes=[
            pltpu.VMEM((B, 3 * OUT), jnp.float32),
            pltpu.VMEM((B, 8), jnp.float32),
        ],
    )(*in_arrays)


def kernel(x_desikan, edge_index_desikan, batch_desikan,
           x_destrieux, edge_index_destrieux, batch_destrieux,
           x_fuzzy, edge_index_fuzzy, batch_fuzzy,
           demographic,
           W1_des, b1_des, W2_des, b2_des,
           W1_det, b1_det, W2_det, b2_det,
           W1_fuz, b1_fuz, W2_fuz, b2_fuz,
           fc1_W, fc1_b, fc2_W, fc2_b, fc3_W, fc3_b):
    xs = (x_desikan, x_destrieux, x_fuzzy)
    eis = (edge_index_desikan, edge_index_destrieux, edge_index_fuzzy)
    bats = (batch_desikan, batch_destrieux, batch_fuzzy)
    w1s, b1s = (W1_des, W1_det, W1_fuz), (b1_des, b1_det, b1_fuz)
    w2s, b2s = (W2_des, W2_det, W2_fuz), (b2_des, b2_det, b2_fuz)

    srcs, dsts, npws = [], [], []
    for ei in eis:
        e_pad = -(-ei.shape[1] // E_ALIGN) * E_ALIGN
        s, d = _pad_edges(ei, e_pad)
        srcs.append(s)
        dsts.append(d)
        npws.append(e_pad // (NW * CH))
    srcs, dsts, npws = tuple(srcs), tuple(dsts), tuple(npws)

    degps = _sc_degrees(dsts, npws)
    degps = [p.reshape(2, NPAD, DW) for p in degps]

    tc1_out = _tc1(xs, w1s, degps)
    h1ps = [tc1_out[0], tc1_out[2], tc1_out[4]]
    dinvs = [tc1_out[1], tc1_out[3], tc1_out[5]]

    s1ps = _sc_aggregate(tuple(h1ps), srcs, dsts, npws, H)
    s1ps = [p.reshape(2, NPAD, H) for p in s1ps]

    h2ps = _tc2(s1ps, h1ps, dinvs, b1s, w2s)

    s2ps = _sc_aggregate(tuple(h2ps), srcs, dsts, npws, OUT)
    s2ps = [p.reshape(2, NPAD, OUT) for p in s2ps]

    branches = []
    for sp, hp, dinv, b2, bat in zip(s2ps, h2ps, dinvs, b2s, bats):
        branches.append((sp, hp, dinv, b2.reshape(1, OUT),
                         bat.reshape(_G, 1, _BLK)))

    return _tc3(branches, demographic, fc1_W, fc1_b.reshape(1, B),
                fc2_W, fc2_b.reshape(1, 32), fc3_W, fc3_b.reshape(1, 2))


# speedup vs baseline: 29.1715x; 1.0369x over previous
import functools

import jax
import jax.numpy as jnp
import numpy as np
from jax import lax
from jax.experimental import pallas as pl
from jax.experimental.pallas import tpu as pltpu
from jax.experimental.pallas import tpu_sc as plsc

N = 10000
B = 64
H = 128
OUT = 64
NC = 2
NS = 16
NW = NC * NS
CH = 128
DUM = 512
NPAD = 10752
RS = NPAD // NS
QB = 8
E_ALIGN = NW * CH * QB
DW = 8

@functools.cache
def _mesh():
    return plsc.VectorSubcoreMesh(core_axis_name="c", subcore_axis_name="s",
                                  num_cores=NC, num_subcores=NS)


def _pad_edges(ei, e_pad):
    e = ei.shape[1]
    k = jnp.arange(e_pad - e, dtype=jnp.int32)
    src = jnp.concatenate([ei[0], k % np.int32(N)]).reshape(e_pad // CH, CH)
    dst = jnp.concatenate([ei[1], np.int32(N) + (k % np.int32(DUM))])
    return src, dst.reshape(e_pad // CH, CH)


def _deg_body(npws, d0, d1, d2, z_ref, ones_ref, o0, o1, o2,
              acc, ones_v, idxd, isem, ssem):
    c = lax.axis_index("c")
    s = lax.axis_index("s")
    w = s * NC + c
    pltpu.sync_copy(ones_ref, ones_v)
    for dst_ref, out_ref, npw in zip((d0, d1, d2), (o0, o1, o2), npws):
        nblk = npw // QB
        pltpu.sync_copy(z_ref.at[pl.ds(s * RS, RS)], acc.at[pl.ds(s * RS, RS)])
        plsc.subcore_barrier()
        row0 = w * npw
        pltpu.sync_copy(dst_ref.at[pl.ds(row0, QB)], idxd.at[0])

        def body(jb, _):
            jm = jb % 2
            jn = (jb + 1) % 2

            @pl.when(jb + 1 < nblk)
            def _():
                pltpu.async_copy(
                    dst_ref.at[pl.ds(row0 + (jb + 1) * QB, QB)],
                    idxd.at[jn], isem)

            for q in range(QB):
                pltpu.async_copy(ones_v, acc.at[idxd.at[jm, q]], ssem,
                                 add=True)
            for q in range(QB):
                pltpu.make_async_copy(ones_v, acc.at[idxd.at[jm, q]],
                                      ssem).wait()

            @pl.when(jb + 1 < nblk)
            def _():
                pltpu.make_async_copy(
                    dst_ref.at[pl.ds(row0, QB)], idxd.at[jn], isem).wait()
            return 0

        lax.fori_loop(0, nblk, body, 0)
        plsc.subcore_barrier()
        pltpu.sync_copy(acc.at[pl.ds(s * RS, RS)],
                        out_ref.at[pl.ds(c * NPAD + s * RS, RS)])
        plsc.subcore_barrier()


def _sc_degrees(dsts, npws):
    z = jnp.zeros((NPAD, DW), jnp.float32)
    ones = jnp.ones((CH, DW), jnp.float32)
    out_t = [jax.ShapeDtypeStruct((2 * NPAD, DW), jnp.float32)] * 3
    fn = pl.kernel(
        functools.partial(_deg_body, tuple(npws)),
        out_type=out_t,
        mesh=_mesh(),
        scratch_types=[
            pltpu.VMEM_SHARED((NPAD, DW), jnp.float32),
            pltpu.VMEM((CH, DW), jnp.float32),
            pltpu.VMEM((2, QB, CH), jnp.int32),
            pltpu.SemaphoreType.DMA,
            pltpu.SemaphoreType.DMA,
        ],
        compiler_params=pltpu.CompilerParams(use_tc_tiling_on_sc=False),
        name="sc_degrees",
    )
    return fn(*dsts, z, ones)


def _agg_body(npws, hd, stage, NS_R, *refs):
    (h0, h1, h2, s0, s1, s2, d0, d1, d2, z_ref, o0, o1, o2,
     acc, idxs, idxd, rows) = refs[:17]
    gsems = refs[17:17 + NS_R]
    ssems = refs[17 + NS_R:17 + 2 * NS_R]
    isem = refs[17 + 2 * NS_R]
    tbl = refs[18 + 2 * NS_R] if stage else None
    c = lax.axis_index("c")
    s = lax.axis_index("s")
    w = s * NC + c
    for h_hbm, src_ref, dst_ref, out_ref, npw in zip(
            (h0, h1, h2), (s0, s1, s2), (d0, d1, d2), (o0, o1, o2), npws):
        nblk = npw // QB
        pltpu.sync_copy(z_ref.at[pl.ds(s * RS, RS)], acc.at[pl.ds(s * RS, RS)])
        if stage:
            pltpu.sync_copy(h_hbm.at[pl.ds(s * (N // NS), N // NS)],
                            tbl.at[pl.ds(s * (N // NS), N // NS)])
            h_ref = tbl
        else:
            h_ref = h_hbm
        plsc.subcore_barrier()

        row0 = w * npw
        pltpu.sync_copy(src_ref.at[pl.ds(row0, QB)], idxs.at[0])
        pltpu.sync_copy(dst_ref.at[pl.ds(row0, QB)], idxd.at[0])
        for p in range(NS_R - 1):
            pltpu.async_copy(h_ref.at[idxs.at[0, p]], rows.at[p], gsems[p])

        def body(jb, _):
            jm = jb % 2
            jn = (jb + 1) % 2

            @pl.when(jb > 0)
            def _():
                pltpu.make_async_copy(
                    rows.at[(QB - 1) % NS_R], acc.at[idxd.at[jn, QB - 1]],
                    ssems[(QB - 1) % NS_R]).wait()

            @pl.when(jb + 1 < nblk)
            def _():
                pltpu.async_copy(
                    src_ref.at[pl.ds(row0 + (jb + 1) * QB, QB)],
                    idxs.at[jn], isem)
                pltpu.async_copy(
                    dst_ref.at[pl.ds(row0 + (jb + 1) * QB, QB)],
                    idxd.at[jn], isem)

            for q in range(QB):
                b = q % NS_R
                bn = (q + NS_R - 1) % NS_R
                pltpu.make_async_copy(
                    h_ref.at[idxs.at[jm, q]], rows.at[b], gsems[b]).wait()
                pltpu.async_copy(rows.at[b], acc.at[idxd.at[jm, q]],
                                 ssems[b], add=True)
                if 0 < q:
                    pltpu.make_async_copy(
                        rows.at[bn], acc.at[idxd.at[jm, q - 1]],
                        ssems[bn]).wait()
                if q + NS_R - 1 < QB:
                    pltpu.async_copy(h_ref.at[idxs.at[jm, q + NS_R - 1]],
                                     rows.at[bn], gsems[bn])
                else:
                    if q == QB - NS_R + 1:
                        @pl.when(jb + 1 < nblk)
                        def _():
                            pltpu.make_async_copy(
                                src_ref.at[pl.ds(row0, QB)], idxs.at[jn],
                                isem).wait()
                            pltpu.make_async_copy(
                                dst_ref.at[pl.ds(row0, QB)], idxd.at[jn],
                                isem).wait()

                    @pl.when(jb + 1 < nblk)
                    def _():
                        pltpu.async_copy(
                            h_ref.at[idxs.at[jn, q + NS_R - 1 - QB]],
                            rows.at[bn], gsems[bn])
            return 0

        lax.fori_loop(0, nblk, body, 0)
        pltpu.make_async_copy(
            rows.at[(QB - 1) % NS_R],
            acc.at[idxd.at[(nblk - 1) % 2, QB - 1]],
            ssems[(QB - 1) % NS_R]).wait()
        plsc.subcore_barrier()
        pltpu.sync_copy(acc.at[pl.ds(s * RS, RS)],
                        out_ref.at[pl.ds(c * NPAD + s * RS, RS)])
        plsc.subcore_barrier()


def _sc_aggregate(hs, srcs, dsts, npws, hd):
    z = jnp.zeros((NPAD, hd), jnp.float32)
    stage = hd * (N + NPAD) * 4 <= 6 * 2**20
    ns_r = 4 if stage else 2
    out_t = [jax.ShapeDtypeStruct((2 * NPAD, hd), jnp.float32)] * 3
    scratch = [
        pltpu.VMEM_SHARED((NPAD, hd), jnp.float32),
        pltpu.VMEM((2, QB, CH), jnp.int32),
        pltpu.VMEM((2, QB, CH), jnp.int32),
        pltpu.VMEM((ns_r, CH, hd), jnp.float32),
    ] + [pltpu.SemaphoreType.DMA] * (2 * ns_r + 1)
    if stage:
        scratch.append(pltpu.VMEM_SHARED((N, hd), jnp.float32))
    fn = pl.kernel(
        functools.partial(_agg_body, tuple(npws), hd, stage, ns_r),
        out_type=out_t,
        mesh=_mesh(),
        scratch_types=scratch,
        compiler_params=pltpu.CompilerParams(use_tc_tiling_on_sc=(hd == H)),
        name=f"sc_gcn_agg_{hd}",
    )
    return fn(*hs, *srcs, *dsts, z)


_BLK = 1000
_G = N // _BLK
_DOT = dict(preferred_element_type=jnp.float32)


def _tc1_body(*refs):
    for b in range(3):
        x_ref, w_ref, degp_ref = refs[3 * b:3 * b + 3]
        hp_ref, dinv_ref = refs[9 + 2 * b:9 + 2 * b + 2]
        deg = degp_ref[0, :, 0:1] + degp_ref[1, :, 0:1] + 1.0
        dinv = lax.rsqrt(deg)
        h = lax.dot_general(x_ref[...], w_ref[...], (((1,), (0,)), ((), ())),
                            **_DOT)
        hp_ref[...] = h * dinv
        dinv_ref[...] = dinv


def _tc1(xs, w1s, degps):
    in_arrays, in_specs = [], []
    for x, w1, degp in zip(xs, w1s, degps):
        d = x.shape[1]
        in_arrays += [x, w1, degp]
        in_specs += [
            pl.BlockSpec((_BLK, d), lambda i: (i, 0)),
            pl.BlockSpec((d, H), lambda i: (0, 0)),
            pl.BlockSpec((2, _BLK, DW), lambda i: (0, i, 0)),
        ]
    return pl.pallas_call(
        _tc1_body,
        grid=(_G,),
        in_specs=in_specs,
        out_specs=[
            pl.BlockSpec((_BLK, H), lambda i: (i, 0)),
            pl.BlockSpec((_BLK, 1), lambda i: (i, 0)),
        ] * 3,
        out_shape=[
            jax.ShapeDtypeStruct((N, H), jnp.float32),
            jax.ShapeDtypeStruct((N, 1), jnp.float32),
        ] * 3,
    )(*in_arrays)


def _tc2_body(*refs):
    for b in range(3):
        sp_ref, hp_ref, dinv_ref, b1_ref, w2_ref = refs[5 * b:5 * b + 5]
        out_ref = refs[15 + b]
        dinv = dinv_ref[...]
        y = (sp_ref[0] + sp_ref[1] + hp_ref[...]) * dinv + b1_ref[...]
        y = jnp.maximum(y, 0.0)
        h2 = lax.dot_general(y, w2_ref[...], (((1,), (0,)), ((), ())), **_DOT)
        out_ref[...] = h2 * dinv


def _tc2(sps, hps, dinvs, b1s, w2s):
    in_arrays, in_specs = [], []
    for sp, hp, dinv, b1, w2 in zip(sps, hps, dinvs, b1s, w2s):
        in_arrays += [sp, hp, dinv, b1.reshape(1, H), w2]
        in_specs += [
            pl.BlockSpec((2, _BLK, H), lambda i: (0, i, 0)),
            pl.BlockSpec((_BLK, H), lambda i: (i, 0)),
            pl.BlockSpec((_BLK, 1), lambda i: (i, 0)),
            pl.BlockSpec((1, H), lambda i: (0, 0)),
            pl.BlockSpec((H, OUT), lambda i: (0, 0)),
        ]
    return pl.pallas_call(
        _tc2_body,
        grid=(_G,),
        in_specs=in_specs,
        out_specs=[pl.BlockSpec((_BLK, OUT), lambda i: (i, 0))] * 3,
        out_shape=[jax.ShapeDtypeStruct((N, OUT), jnp.float32)] * 3,
    )(*in_arrays)


def _tc3_body(sp0, hp0, di0, bb0, bt0,
              sp1, hp1, di1, bb1, bt1,
              sp2, hp2, di2, bb2, bt2,
              demo_ref, f1w, f1b, f2w, f2b, f3w, f3b,
              out_ref, pooled, counts):
    i = pl.program_id(0)

    @pl.when(i == 0)
    def _():
        pooled[...] = jnp.zeros_like(pooled)
        counts[...] = jnp.zeros_like(counts)

    ones_col = jnp.ones((_BLK, 1), jnp.float32)
    for b, (sp, hp, di, bb, bt) in enumerate((
            (sp0, hp0, di0, bb0, bt0),
            (sp1, hp1, di1, bb1, bt1),
            (sp2, hp2, di2, bb2, bt2))):
        y = (sp[0] + sp[1] + hp[...]) * di[...] + bb[...]
        y = jnp.maximum(y, 0.0)
        gids = lax.broadcasted_iota(jnp.int32, (B, _BLK), 0)
        m = (gids == bt[0]).astype(jnp.float32)
        pooled[:, b * OUT:(b + 1) * OUT] += lax.dot_general(
            m, y, (((1,), (0,)), ((), ())), **_DOT)
        counts[:, b:b + 1] += lax.dot_general(
            m, ones_col, (((1,), (0,)), ((), ())), **_DOT)

    @pl.when(i == _G - 1)
    def _():
        cnt = jnp.maximum(counts[...], 1.0)
        h = f1b[...]
        for b in range(3):
            p = pooled[:, b * OUT:(b + 1) * OUT] / cnt[:, b:b + 1]
            h = h + lax.dot_general(
                p, f1w[b * OUT:(b + 1) * OUT, :],
                (((1,), (0,)), ((), ())), **_DOT)
        h = h + lax.dot_general(demo_ref[...], f1w[3 * OUT:, :],
                                (((1,), (0,)), ((), ())), **_DOT)
        h = jnp.maximum(h, 0.0)
        h = jnp.maximum(lax.dot_general(h, f2w[...],
                                        (((1,), (0,)), ((), ())), **_DOT)
                        + f2b[...], 0.0)
        out_ref[...] = lax.dot_general(h, f3w[...],
                                       (((1,), (0,)), ((), ())), **_DOT) \
            + f3b[...]


def _tc3(branches, demo, f1w, f1b, f2w, f2b, f3w, f3b):
    in_arrays = []
    in_specs = []
    for sp, hp, dinv, b2, bat3 in branches:
        in_arrays += [sp, hp, dinv, b2, bat3]
        in_specs += [
            pl.BlockSpec((2, _BLK, OUT), lambda i: (0, i, 0)),
            pl.BlockSpec((_BLK, OUT), lambda i: (i, 0)),
            pl.BlockSpec((_BLK, 1), lambda i: (i, 0)),
            pl.BlockSpec((1, OUT), lambda i: (0, 0)),
            pl.BlockSpec((1, 1, _BLK), lambda i: (i, 0, 0)),
        ]
    in_arrays += [demo, f1w, f1b, f2w, f2b, f3w, f3b]
    in_specs += [
        pl.BlockSpec((B, 16), lambda i: (0, 0)),
        pl.BlockSpec((3 * OUT + 16, B), lambda i: (0, 0)),
        pl.BlockSpec((1, B), lambda i: (0, 0)),
        pl.BlockSpec((B, 32), lambda i: (0, 0)),
        pl.BlockSpec((1, 32), lambda i: (0, 0)),
        pl.BlockSpec((32, 2), lambda i: (0, 0)),
        pl.BlockSpec((1, 2), lambda i: (0, 0)),
    ]
    return pl.pallas_call(
        _tc3_body,
        grid=(_G,),
        in_specs=in_specs,
        out_specs=pl.BlockSpec((B, 2), lambda i: (0, 0)),
        out_shape=jax.ShapeDtypeStruct((B, 2), jnp.float32),
        scratch_shapes=[
            pltpu.VMEM((B, 3 * OUT), jnp.float32),
            pltpu.VMEM((B, 8), jnp.float32),
        ],
    )(*in_arrays)


def kernel(x_desikan, edge_index_desikan, batch_desikan,
           x_destrieux, edge_index_destrieux, batch_destrieux,
           x_fuzzy, edge_index_fuzzy, batch_fuzzy,
           demographic,
           W1_des, b1_des, W2_des, b2_des,
           W1_det, b1_det, W2_det, b2_det,
           W1_fuz, b1_fuz, W2_fuz, b2_fuz,
           fc1_W, fc1_b, fc2_W, fc2_b, fc3_W, fc3_b):
    xs = (x_desikan, x_destrieux, x_fuzzy)
    eis = (edge_index_desikan, edge_index_destrieux, edge_index_fuzzy)
    bats = (batch_desikan, batch_destrieux, batch_fuzzy)
    w1s, b1s = (W1_des, W1_det, W1_fuz), (b1_des, b1_det, b1_fuz)
    w2s, b2s = (W2_des, W2_det, W2_fuz), (b2_des, b2_det, b2_fuz)

    srcs, dsts, npws = [], [], []
    for ei in eis:
        e_pad = -(-ei.shape[1] // E_ALIGN) * E_ALIGN
        s, d = _pad_edges(ei, e_pad)
        srcs.append(s)
        dsts.append(d)
        npws.append(e_pad // (NW * CH))
    srcs, dsts, npws = tuple(srcs), tuple(dsts), tuple(npws)

    degps = _sc_degrees(dsts, npws)
    degps = [p.reshape(2, NPAD, DW) for p in degps]

    tc1_out = _tc1(xs, w1s, degps)
    h1ps = [tc1_out[0], tc1_out[2], tc1_out[4]]
    dinvs = [tc1_out[1], tc1_out[3], tc1_out[5]]

    s1ps = _sc_aggregate(tuple(h1ps), srcs, dsts, npws, H)
    s1ps = [p.reshape(2, NPAD, H) for p in s1ps]

    h2ps = _tc2(s1ps, h1ps, dinvs, b1s, w2s)

    s2ps = _sc_aggregate(tuple(h2ps), srcs, dsts, npws, OUT)
    s2ps = [p.reshape(2, NPAD, OUT) for p in s2ps]

    branches = []
    for sp, hp, dinv, b2, bat in zip(s2ps, h2ps, dinvs, b2s, bats):
        branches.append((sp, hp, dinv, b2.reshape(1, OUT),
                         bat.reshape(_G, 1, _BLK)))

    return _tc3(branches, demographic, fc1_W, fc1_b.reshape(1, B),
                fc2_W, fc2_b.reshape(1, 32), fc3_W, fc3_b.reshape(1, 2))


# speedup vs baseline: 30.0240x; 1.0292x over previous
import functools

import jax
import jax.numpy as jnp
import numpy as np
from jax import lax
from jax.experimental import pallas as pl
from jax.experimental.pallas import tpu as pltpu
from jax.experimental.pallas import tpu_sc as plsc

N = 10000
B = 64
H = 128
OUT = 64
NC = 2
NS = 16
NW = NC * NS
CH = 128
DUM = 512
NPAD = 10752
RS = NPAD // NS
QB = 8
E_ALIGN = NW * CH * QB
DW = 8

@functools.cache
def _mesh():
    return plsc.VectorSubcoreMesh(core_axis_name="c", subcore_axis_name="s",
                                  num_cores=NC, num_subcores=NS)


_EPB = 256 * CH


def _edgeprep_body(es, e_pads, *refs):
    i = pl.program_id(0)
    eis, outs = refs[:3], refs[3:]
    for b, (ei_ref, e, e_pad) in enumerate(zip(eis, es, e_pads)):
        nblk = e_pad // _EPB
        bi = jnp.minimum(i, nblk - 1)
        g = (bi * _EPB
             + CH * lax.broadcasted_iota(jnp.int32, (_EPB // CH, CH), 0)
             + lax.broadcasted_iota(jnp.int32, (_EPB // CH, CH), 1))
        mask = g < e
        s2 = ei_ref[0].reshape(_EPB // CH, CH)
        d2 = ei_ref[1].reshape(_EPB // CH, CH)
        outs[2 * b][...] = jnp.where(mask, s2, g % np.int32(N))
        outs[2 * b + 1][...] = jnp.where(
            mask, d2, np.int32(N) + g % np.int32(DUM))


def _edge_prep(eis, e_pads):
    gmax = max(e_pads) // _EPB
    in_specs, out_specs, out_shape = [], [], []
    for ei, e_pad in zip(eis, e_pads):
        nblk = e_pad // _EPB

        def imap(i, nblk=nblk):
            return (0, jnp.minimum(i, nblk - 1))

        def omap(i, nblk=nblk):
            return (jnp.minimum(i, nblk - 1), 0)

        in_specs.append(pl.BlockSpec((2, _EPB), imap))
        out_specs += [pl.BlockSpec((_EPB // CH, CH), omap)] * 2
        out_shape += [jax.ShapeDtypeStruct((e_pad // CH, CH), jnp.int32)] * 2
    res = pl.pallas_call(
        functools.partial(_edgeprep_body,
                          tuple(ei.shape[1] for ei in eis), tuple(e_pads)),
        grid=(gmax,),
        in_specs=in_specs,
        out_specs=out_specs,
        out_shape=out_shape,
    )(*eis)
    return res[0::2], res[1::2]


def _deg_body(npws, d0, d1, d2, z_ref, ones_ref, o0, o1, o2,
              acc, ones_v, idxd, isem, ssem):
    c = lax.axis_index("c")
    s = lax.axis_index("s")
    w = s * NC + c
    pltpu.sync_copy(ones_ref, ones_v)
    for dst_ref, out_ref, npw in zip((d0, d1, d2), (o0, o1, o2), npws):
        nblk = npw // QB
        pltpu.sync_copy(z_ref.at[pl.ds(s * RS, RS)], acc.at[pl.ds(s * RS, RS)])
        plsc.subcore_barrier()
        row0 = w * npw
        pltpu.sync_copy(dst_ref.at[pl.ds(row0, QB)], idxd.at[0])

        def body(jb, _):
            jm = jb % 2
            jn = (jb + 1) % 2

            @pl.when(jb + 1 < nblk)
            def _():
                pltpu.async_copy(
                    dst_ref.at[pl.ds(row0 + (jb + 1) * QB, QB)],
                    idxd.at[jn], isem)

            for q in range(QB):
                pltpu.async_copy(ones_v, acc.at[idxd.at[jm, q]], ssem,
                                 add=True)
            for q in range(QB):
                pltpu.make_async_copy(ones_v, acc.at[idxd.at[jm, q]],
                                      ssem).wait()

            @pl.when(jb + 1 < nblk)
            def _():
                pltpu.make_async_copy(
                    dst_ref.at[pl.ds(row0, QB)], idxd.at[jn], isem).wait()
            return 0

        lax.fori_loop(0, nblk, body, 0)
        plsc.subcore_barrier()
        pltpu.sync_copy(acc.at[pl.ds(s * RS, RS)],
                        out_ref.at[pl.ds(c * NPAD + s * RS, RS)])
        plsc.subcore_barrier()


def _sc_degrees(dsts, npws):
    z = jnp.zeros((NPAD, DW), jnp.float32)
    ones = jnp.ones((CH, DW), jnp.float32)
    out_t = [jax.ShapeDtypeStruct((2 * NPAD, DW), jnp.float32)] * 3
    fn = pl.kernel(
        functools.partial(_deg_body, tuple(npws)),
        out_type=out_t,
        mesh=_mesh(),
        scratch_types=[
            pltpu.VMEM_SHARED((NPAD, DW), jnp.float32),
            pltpu.VMEM((CH, DW), jnp.float32),
            pltpu.VMEM((2, QB, CH), jnp.int32),
            pltpu.SemaphoreType.DMA,
            pltpu.SemaphoreType.DMA,
        ],
        compiler_params=pltpu.CompilerParams(use_tc_tiling_on_sc=False),
        name="sc_degrees",
    )
    return fn(*dsts, z, ones)


def _agg_body(npws, hd, stage, NS_R, *refs):
    (h0, h1, h2, s0, s1, s2, d0, d1, d2, z_ref, o0, o1, o2,
     acc, idxs, idxd, rows) = refs[:17]
    gsems = refs[17:17 + NS_R]
    ssems = refs[17 + NS_R:17 + 2 * NS_R]
    isem = refs[17 + 2 * NS_R]
    tbl = refs[18 + 2 * NS_R] if stage else None
    c = lax.axis_index("c")
    s = lax.axis_index("s")
    w = s * NC + c
    for h_hbm, src_ref, dst_ref, out_ref, npw in zip(
            (h0, h1, h2), (s0, s1, s2), (d0, d1, d2), (o0, o1, o2), npws):
        nblk = npw // QB
        pltpu.sync_copy(z_ref.at[pl.ds(s * RS, RS)], acc.at[pl.ds(s * RS, RS)])
        if stage:
            pltpu.sync_copy(h_hbm.at[pl.ds(s * 624, 624)],
                            tbl.at[pl.ds(s * 624, 624)])

            @pl.when(s == 0)
            def _():
                pltpu.sync_copy(h_hbm.at[pl.ds(16 * 624, N - 16 * 624)],
                                tbl.at[pl.ds(16 * 624, N - 16 * 624)])
            h_ref = tbl
        else:
            h_ref = h_hbm
        plsc.subcore_barrier()

        row0 = w * npw
        pltpu.sync_copy(src_ref.at[pl.ds(row0, QB)], idxs.at[0])
        pltpu.sync_copy(dst_ref.at[pl.ds(row0, QB)], idxd.at[0])
        for p in range(NS_R - 1):
            pltpu.async_copy(h_ref.at[idxs.at[0, p]], rows.at[p], gsems[p])

        def body(jb, _):
            jm = jb % 2
            jn = (jb + 1) % 2

            @pl.when(jb > 0)
            def _():
                pltpu.make_async_copy(
                    rows.at[(QB - 1) % NS_R], acc.at[idxd.at[jn, QB - 1]],
                    ssems[(QB - 1) % NS_R]).wait()

            @pl.when(jb + 1 < nblk)
            def _():
                pltpu.async_copy(
                    src_ref.at[pl.ds(row0 + (jb + 1) * QB, QB)],
                    idxs.at[jn], isem)
                pltpu.async_copy(
                    dst_ref.at[pl.ds(row0 + (jb + 1) * QB, QB)],
                    idxd.at[jn], isem)

            for q in range(QB):
                b = q % NS_R
                bn = (q + NS_R - 1) % NS_R
                pltpu.make_async_copy(
                    h_ref.at[idxs.at[jm, q]], rows.at[b], gsems[b]).wait()
                pltpu.async_copy(rows.at[b], acc.at[idxd.at[jm, q]],
                                 ssems[b], add=True)
                if 0 < q:
                    pltpu.make_async_copy(
                        rows.at[bn], acc.at[idxd.at[jm, q - 1]],
                        ssems[bn]).wait()
                if q + NS_R - 1 < QB:
                    pltpu.async_copy(h_ref.at[idxs.at[jm, q + NS_R - 1]],
                                     rows.at[bn], gsems[bn])
                else:
                    if q == QB - NS_R + 1:
                        @pl.when(jb + 1 < nblk)
                        def _():
                            pltpu.make_async_copy(
                                src_ref.at[pl.ds(row0, QB)], idxs.at[jn],
                                isem).wait()
                            pltpu.make_async_copy(
                                dst_ref.at[pl.ds(row0, QB)], idxd.at[jn],
                                isem).wait()

                    @pl.when(jb + 1 < nblk)
                    def _():
                        pltpu.async_copy(
                            h_ref.at[idxs.at[jn, q + NS_R - 1 - QB]],
                            rows.at[bn], gsems[bn])
            return 0

        lax.fori_loop(0, nblk, body, 0)
        pltpu.make_async_copy(
            rows.at[(QB - 1) % NS_R],
            acc.at[idxd.at[(nblk - 1) % 2, QB - 1]],
            ssems[(QB - 1) % NS_R]).wait()
        plsc.subcore_barrier()
        pltpu.sync_copy(acc.at[pl.ds(s * RS, RS)],
                        out_ref.at[pl.ds(c * NPAD + s * RS, RS)])
        plsc.subcore_barrier()


def _sc_aggregate(hs, srcs, dsts, npws, hd):
    z = jnp.zeros((NPAD, hd), jnp.float32)
    stage = hd * (N + NPAD) * 4 <= 6 * 2**20
    ns_r = 4 if stage else 2
    out_t = [jax.ShapeDtypeStruct((2 * NPAD, hd), jnp.float32)] * 3
    scratch = [
        pltpu.VMEM_SHARED((NPAD, hd), jnp.float32),
        pltpu.VMEM((2, QB, CH), jnp.int32),
        pltpu.VMEM((2, QB, CH), jnp.int32),
        pltpu.VMEM((ns_r, CH, hd), jnp.float32),
    ] + [pltpu.SemaphoreType.DMA] * (2 * ns_r + 1)
    if stage:
        scratch.append(pltpu.VMEM_SHARED((N, hd), jnp.float32))
    fn = pl.kernel(
        functools.partial(_agg_body, tuple(npws), hd, stage, ns_r),
        out_type=out_t,
        mesh=_mesh(),
        scratch_types=scratch,
        compiler_params=pltpu.CompilerParams(use_tc_tiling_on_sc=(hd == H)),
        name=f"sc_gcn_agg_{hd}",
    )
    return fn(*hs, *srcs, *dsts, z)


_BLK = 1000
_G = N // _BLK
_DOT = dict(preferred_element_type=jnp.float32)


def _tc1a_body(*refs):
    for b in range(3):
        x_ref, w_ref = refs[2 * b:2 * b + 2]
        refs[6 + b][...] = lax.dot_general(
            x_ref[...], w_ref[...], (((1,), (0,)), ((), ())), **_DOT)


def _tc1a(xs, w1s):
    in_arrays, in_specs = [], []
    for x, w1 in zip(xs, w1s):
        d = x.shape[1]
        in_arrays += [x, w1]
        in_specs += [
            pl.BlockSpec((_BLK, d), lambda i: (i, 0)),
            pl.BlockSpec((d, H), lambda i: (0, 0)),
        ]
    return pl.pallas_call(
        _tc1a_body,
        grid=(_G,),
        in_specs=in_specs,
        out_specs=[pl.BlockSpec((_BLK, H), lambda i: (i, 0))] * 3,
        out_shape=[jax.ShapeDtypeStruct((N, H), jnp.float32)] * 3,
    )(*in_arrays)


def _tc1b_body(*refs):
    for b in range(3):
        h_ref, degp_ref = refs[2 * b:2 * b + 2]
        hp_ref, dinv_ref = refs[6 + 2 * b:6 + 2 * b + 2]
        deg = degp_ref[0, :, 0:1] + degp_ref[1, :, 0:1] + 1.0
        dinv = lax.rsqrt(deg)
        hp_ref[...] = h_ref[...] * dinv
        dinv_ref[...] = dinv


def _tc1b(hs, degps):
    in_arrays, in_specs = [], []
    for h, degp in zip(hs, degps):
        in_arrays += [h, degp]
        in_specs += [
            pl.BlockSpec((_BLK, H), lambda i: (i, 0)),
            pl.BlockSpec((2, _BLK, DW), lambda i: (0, i, 0)),
        ]
    return pl.pallas_call(
        _tc1b_body,
        grid=(_G,),
        in_specs=in_specs,
        out_specs=[
            pl.BlockSpec((_BLK, H), lambda i: (i, 0)),
            pl.BlockSpec((_BLK, 1), lambda i: (i, 0)),
        ] * 3,
        out_shape=[
            jax.ShapeDtypeStruct((N, H), jnp.float32),
            jax.ShapeDtypeStruct((N, 1), jnp.float32),
        ] * 3,
    )(*in_arrays)


def _tc2_body(*refs):
    for b in range(3):
        sp_ref, hp_ref, dinv_ref, b1_ref, w2_ref = refs[5 * b:5 * b + 5]
        out_ref = refs[15 + b]
        dinv = dinv_ref[...]
        y = (sp_ref[0] + sp_ref[1] + hp_ref[...]) * dinv + b1_ref[...]
        y = jnp.maximum(y, 0.0)
        h2 = lax.dot_general(y, w2_ref[...], (((1,), (0,)), ((), ())), **_DOT)
        out_ref[...] = h2 * dinv


def _tc2(sps, hps, dinvs, b1s, w2s):
    in_arrays, in_specs = [], []
    for sp, hp, dinv, b1, w2 in zip(sps, hps, dinvs, b1s, w2s):
        in_arrays += [sp, hp, dinv, b1.reshape(1, H), w2]
        in_specs += [
            pl.BlockSpec((2, _BLK, H), lambda i: (0, i, 0)),
            pl.BlockSpec((_BLK, H), lambda i: (i, 0)),
            pl.BlockSpec((_BLK, 1), lambda i: (i, 0)),
            pl.BlockSpec((1, H), lambda i: (0, 0)),
            pl.BlockSpec((H, OUT), lambda i: (0, 0)),
        ]
    return pl.pallas_call(
        _tc2_body,
        grid=(_G,),
        in_specs=in_specs,
        out_specs=[pl.BlockSpec((_BLK, OUT), lambda i: (i, 0))] * 3,
        out_shape=[jax.ShapeDtypeStruct((N, OUT), jnp.float32)] * 3,
    )(*in_arrays)


def _tc3_body(sp0, hp0, di0, bb0, bt0,
              sp1, hp1, di1, bb1, bt1,
              sp2, hp2, di2, bb2, bt2,
              demo_ref, f1w, f1b, f2w, f2b, f3w, f3b,
              out_ref, pooled, counts):
    i = pl.program_id(0)

    @pl.when(i == 0)
    def _():
        pooled[...] = jnp.zeros_like(pooled)
        counts[...] = jnp.zeros_like(counts)

    ones_col = jnp.ones((_BLK, 1), jnp.float32)
    for b, (sp, hp, di, bb, bt) in enumerate((
            (sp0, hp0, di0, bb0, bt0),
            (sp1, hp1, di1, bb1, bt1),
            (sp2, hp2, di2, bb2, bt2))):
        y = (sp[0] + sp[1] + hp[...]) * di[...] + bb[...]
        y = jnp.maximum(y, 0.0)
        gids = lax.broadcasted_iota(jnp.int32, (B, _BLK), 0)
        m = (gids == bt[0]).astype(jnp.float32)
        pooled[:, b * OUT:(b + 1) * OUT] += lax.dot_general(
            m, y, (((1,), (0,)), ((), ())), **_DOT)
        counts[:, b:b + 1] += lax.dot_general(
            m, ones_col, (((1,), (0,)), ((), ())), **_DOT)

    @pl.when(i == _G - 1)
    def _():
        cnt = jnp.maximum(counts[...], 1.0)
        h = f1b[...]
        for b in range(3):
            p = pooled[:, b * OUT:(b + 1) * OUT] / cnt[:, b:b + 1]
            h = h + lax.dot_general(
                p, f1w[b * OUT:(b + 1) * OUT, :],
                (((1,), (0,)), ((), ())), **_DOT)
        h = h + lax.dot_general(demo_ref[...], f1w[3 * OUT:, :],
                                (((1,), (0,)), ((), ())), **_DOT)
        h = jnp.maximum(h, 0.0)
        h = jnp.maximum(lax.dot_general(h, f2w[...],
                                        (((1,), (0,)), ((), ())), **_DOT)
                        + f2b[...], 0.0)
        out_ref[...] = lax.dot_general(h, f3w[...],
                                       (((1,), (0,)), ((), ())), **_DOT) \
            + f3b[...]


def _tc3(branches, demo, f1w, f1b, f2w, f2b, f3w, f3b):
    in_arrays = []
    in_specs = []
    for sp, hp, dinv, b2, bat3 in branches:
        in_arrays += [sp, hp, dinv, b2, bat3]
        in_specs += [
            pl.BlockSpec((2, _BLK, OUT), lambda i: (0, i, 0)),
            pl.BlockSpec((_BLK, OUT), lambda i: (i, 0)),
            pl.BlockSpec((_BLK, 1), lambda i: (i, 0)),
            pl.BlockSpec((1, OUT), lambda i: (0, 0)),
            pl.BlockSpec((1, 1, _BLK), lambda i: (i, 0, 0)),
        ]
    in_arrays += [demo, f1w, f1b, f2w, f2b, f3w, f3b]
    in_specs += [
        pl.BlockSpec((B, 16), lambda i: (0, 0)),
        pl.BlockSpec((3 * OUT + 16, B), lambda i: (0, 0)),
        pl.BlockSpec((1, B), lambda i: (0, 0)),
        pl.BlockSpec((B, 32), lambda i: (0, 0)),
        pl.BlockSpec((1, 32), lambda i: (0, 0)),
        pl.BlockSpec((32, 2), lambda i: (0, 0)),
        pl.BlockSpec((1, 2), lambda i: (0, 0)),
    ]
    return pl.pallas_call(
        _tc3_body,
        grid=(_G,),
        in_specs=in_specs,
        out_specs=pl.BlockSpec((B, 2), lambda i: (0, 0)),
        out_shape=jax.ShapeDtypeStruct((B, 2), jnp.float32),
        scratch_shapes=[
            pltpu.VMEM((B, 3 * OUT), jnp.float32),
            pltpu.VMEM((B, 8), jnp.float32),
        ],
    )(*in_arrays)


def kernel(x_desikan, edge_index_desikan, batch_desikan,
           x_destrieux, edge_index_destrieux, batch_destrieux,
           x_fuzzy, edge_index_fuzzy, batch_fuzzy,
           demographic,
           W1_des, b1_des, W2_des, b2_des,
           W1_det, b1_det, W2_det, b2_det,
           W1_fuz, b1_fuz, W2_fuz, b2_fuz,
           fc1_W, fc1_b, fc2_W, fc2_b, fc3_W, fc3_b):
    xs = (x_desikan, x_destrieux, x_fuzzy)
    eis = (edge_index_desikan, edge_index_destrieux, edge_index_fuzzy)
    bats = (batch_desikan, batch_destrieux, batch_fuzzy)
    w1s, b1s = (W1_des, W1_det, W1_fuz), (b1_des, b1_det, b1_fuz)
    w2s, b2s = (W2_des, W2_det, W2_fuz), (b2_des, b2_det, b2_fuz)

    e_pads = [-(-ei.shape[1] // E_ALIGN) * E_ALIGN for ei in eis]
    npws = tuple(e_pad // (NW * CH) for e_pad in e_pads)
    srcs, dsts = _edge_prep(eis, e_pads)
    srcs, dsts = tuple(srcs), tuple(dsts)

    degps = _sc_degrees(dsts, npws)
    degps = [p.reshape(2, NPAD, DW) for p in degps]

    h1s = _tc1a(xs, w1s)
    tc1_out = _tc1b(h1s, degps)
    h1ps = [tc1_out[0], tc1_out[2], tc1_out[4]]
    dinvs = [tc1_out[1], tc1_out[3], tc1_out[5]]

    s1ps = _sc_aggregate(tuple(h1ps), srcs, dsts, npws, H)
    s1ps = [p.reshape(2, NPAD, H) for p in s1ps]

    h2ps = _tc2(s1ps, h1ps, dinvs, b1s, w2s)

    s2ps = _sc_aggregate(tuple(h2ps), srcs, dsts, npws, OUT)
    s2ps = [p.reshape(2, NPAD, OUT) for p in s2ps]

    branches = []
    for sp, hp, dinv, b2, bat in zip(s2ps, h2ps, dinvs, b2s, bats):
        branches.append((sp, hp, dinv, b2.reshape(1, OUT),
                         bat.reshape(_G, 1, _BLK)))

    return _tc3(branches, demographic, fc1_W, fc1_b.reshape(1, B),
                fc2_W, fc2_b.reshape(1, 32), fc3_W, fc3_b.reshape(1, 2))


# speedup vs baseline: 31.4045x; 1.0460x over previous
import functools

import jax
import jax.numpy as jnp
import numpy as np
from jax import lax
from jax.experimental import pallas as pl
from jax.experimental.pallas import tpu as pltpu
from jax.experimental.pallas import tpu_sc as plsc

N = 10000
B = 64
H = 128
OUT = 64
NC = 2
NS = 16
NW = NC * NS
CH = 128
DUM = 512
NPAD = 10752
RS = NPAD // NS
QB = 8
E_ALIGN = NW * CH * QB
DW = 8

@functools.cache
def _mesh():
    return plsc.VectorSubcoreMesh(core_axis_name="c", subcore_axis_name="s",
                                  num_cores=NC, num_subcores=NS)


_EPB = 256 * CH


def _edgeprep_body(es, e_pads, *refs):
    i = pl.program_id(0)
    eis, outs = refs[:3], refs[3:]
    for b, (ei_ref, e, e_pad) in enumerate(zip(eis, es, e_pads)):
        nblk = e_pad // _EPB
        bi = jnp.minimum(i, nblk - 1)
        g = (bi * _EPB
             + CH * lax.broadcasted_iota(jnp.int32, (_EPB // CH, CH), 0)
             + lax.broadcasted_iota(jnp.int32, (_EPB // CH, CH), 1))
        mask = g < e
        s2 = ei_ref[0].reshape(_EPB // CH, CH)
        d2 = ei_ref[1].reshape(_EPB // CH, CH)
        outs[2 * b][...] = jnp.where(mask, s2, g % np.int32(N))
        outs[2 * b + 1][...] = jnp.where(
            mask, d2, np.int32(N) + g % np.int32(DUM))


def _edge_prep(eis, e_pads):
    gmax = max(e_pads) // _EPB
    in_specs, out_specs, out_shape = [], [], []
    for ei, e_pad in zip(eis, e_pads):
        nblk = e_pad // _EPB

        def imap(i, nblk=nblk):
            return (0, jnp.minimum(i, nblk - 1))

        def omap(i, nblk=nblk):
            return (jnp.minimum(i, nblk - 1), 0)

        in_specs.append(pl.BlockSpec((2, _EPB), imap))
        out_specs += [pl.BlockSpec((_EPB // CH, CH), omap)] * 2
        out_shape += [jax.ShapeDtypeStruct((e_pad // CH, CH), jnp.int32)] * 2
    res = pl.pallas_call(
        functools.partial(_edgeprep_body,
                          tuple(ei.shape[1] for ei in eis), tuple(e_pads)),
        grid=(gmax,),
        in_specs=in_specs,
        out_specs=out_specs,
        out_shape=out_shape,
    )(*eis)
    return res[0::2], res[1::2]


def _deg_body(npws, d0, d1, d2, z_ref, ones_ref, o0, o1, o2,
              acc, ones_v, idxd, isem, ssem):
    c = lax.axis_index("c")
    s = lax.axis_index("s")
    w = s * NC + c
    pltpu.sync_copy(ones_ref, ones_v)
    for dst_ref, out_ref, npw in zip((d0, d1, d2), (o0, o1, o2), npws):
        nblk = npw // QB
        pltpu.sync_copy(z_ref.at[pl.ds(s * RS, RS)], acc.at[pl.ds(s * RS, RS)])
        plsc.subcore_barrier()
        row0 = w * npw
        pltpu.sync_copy(dst_ref.at[pl.ds(row0, QB)], idxd.at[0])

        def body(jb, _):
            jm = jb % 2
            jn = (jb + 1) % 2

            @pl.when(jb + 1 < nblk)
            def _():
                pltpu.async_copy(
                    dst_ref.at[pl.ds(row0 + (jb + 1) * QB, QB)],
                    idxd.at[jn], isem)

            for q in range(QB):
                pltpu.async_copy(ones_v, acc.at[idxd.at[jm, q]], ssem,
                                 add=True)
            for q in range(QB):
                pltpu.make_async_copy(ones_v, acc.at[idxd.at[jm, q]],
                                      ssem).wait()

            @pl.when(jb + 1 < nblk)
            def _():
                pltpu.make_async_copy(
                    dst_ref.at[pl.ds(row0, QB)], idxd.at[jn], isem).wait()
            return 0

        lax.fori_loop(0, nblk, body, 0)
        plsc.subcore_barrier()
        pltpu.sync_copy(acc.at[pl.ds(s * RS, RS)],
                        out_ref.at[pl.ds(c * NPAD + s * RS, RS)])
        plsc.subcore_barrier()


def _sc_degrees(dsts, npws):
    z = jnp.zeros((NPAD, DW), jnp.float32)
    ones = jnp.ones((CH, DW), jnp.float32)
    out_t = [jax.ShapeDtypeStruct((2 * NPAD, DW), jnp.float32)] * 3
    fn = pl.kernel(
        functools.partial(_deg_body, tuple(npws)),
        out_type=out_t,
        mesh=_mesh(),
        scratch_types=[
            pltpu.VMEM_SHARED((NPAD, DW), jnp.float32),
            pltpu.VMEM((CH, DW), jnp.float32),
            pltpu.VMEM((2, QB, CH), jnp.int32),
            pltpu.SemaphoreType.DMA,
            pltpu.SemaphoreType.DMA,
        ],
        compiler_params=pltpu.CompilerParams(use_tc_tiling_on_sc=False),
        name="sc_degrees",
    )
    return fn(*dsts, z, ones)


def _agg_body(npws, hd, stage, NS_R, *refs):
    hs, srcs, dsts, z_ref = refs[0:3], refs[3:6], refs[6:9], refs[9]
    nout = 6 if stage else 3
    of = refs[10:10 + nout]
    outs = [tuple(of[2 * b:2 * b + 2]) if stage else of[b] for b in range(3)]
    rest = refs[10 + nout:]
    acc, idxs, idxd, rows = rest[0:4]
    gsems = rest[4:4 + NS_R]
    ssems = rest[4 + NS_R:4 + 2 * NS_R]
    isem = rest[4 + 2 * NS_R]
    tbl = rest[5 + 2 * NS_R] if stage else None
    c = lax.axis_index("c")
    s = lax.axis_index("s")
    w = s * NC + c
    for h_hbm, src_ref, dst_ref, out_ref, npw in zip(
            hs, srcs, dsts, outs, npws):
        nblk = npw // QB
        pltpu.sync_copy(z_ref.at[pl.ds(s * RS, RS)], acc.at[pl.ds(s * RS, RS)])
        if stage:
            pltpu.sync_copy(h_hbm.at[pl.ds(s * 624, 624)],
                            tbl.at[pl.ds(s * 624, 624)])

            @pl.when(s == 0)
            def _():
                pltpu.sync_copy(h_hbm.at[pl.ds(16 * 624, N - 16 * 624)],
                                tbl.at[pl.ds(16 * 624, N - 16 * 624)])
            h_ref = tbl
        else:
            h_ref = h_hbm
        plsc.subcore_barrier()

        row0 = w * npw
        pltpu.sync_copy(src_ref.at[pl.ds(row0, QB)], idxs.at[0])
        pltpu.sync_copy(dst_ref.at[pl.ds(row0, QB)], idxd.at[0])
        for p in range(NS_R - 1):
            pltpu.async_copy(h_ref.at[idxs.at[0, p]], rows.at[p], gsems[p])

        def body(jb, _):
            jm = jb % 2
            jn = (jb + 1) % 2

            @pl.when(jb > 0)
            def _():
                pltpu.make_async_copy(
                    rows.at[(QB - 1) % NS_R], acc.at[idxd.at[jn, QB - 1]],
                    ssems[(QB - 1) % NS_R]).wait()

            @pl.when(jb + 1 < nblk)
            def _():
                pltpu.async_copy(
                    src_ref.at[pl.ds(row0 + (jb + 1) * QB, QB)],
                    idxs.at[jn], isem)
                pltpu.async_copy(
                    dst_ref.at[pl.ds(row0 + (jb + 1) * QB, QB)],
                    idxd.at[jn], isem)

            for q in range(QB):
                b = q % NS_R
                bn = (q + NS_R - 1) % NS_R
                pltpu.make_async_copy(
                    h_ref.at[idxs.at[jm, q]], rows.at[b], gsems[b]).wait()
                pltpu.async_copy(rows.at[b], acc.at[idxd.at[jm, q]],
                                 ssems[b], add=True)
                if 0 < q:
                    pltpu.make_async_copy(
                        rows.at[bn], acc.at[idxd.at[jm, q - 1]],
                        ssems[bn]).wait()
                if q + NS_R - 1 < QB:
                    pltpu.async_copy(h_ref.at[idxs.at[jm, q + NS_R - 1]],
                                     rows.at[bn], gsems[bn])
                else:
                    if q == QB - NS_R + 1:
                        @pl.when(jb + 1 < nblk)
                        def _():
                            pltpu.make_async_copy(
                                src_ref.at[pl.ds(row0, QB)], idxs.at[jn],
                                isem).wait()
                            pltpu.make_async_copy(
                                dst_ref.at[pl.ds(row0, QB)], idxd.at[jn],
                                isem).wait()

                    @pl.when(jb + 1 < nblk)
                    def _():
                        pltpu.async_copy(
                            h_ref.at[idxs.at[jn, q + NS_R - 1 - QB]],
                            rows.at[bn], gsems[bn])
            return 0

        lax.fori_loop(0, nblk, body, 0)
        pltpu.make_async_copy(
            rows.at[(QB - 1) % NS_R],
            acc.at[idxd.at[(nblk - 1) % 2, QB - 1]],
            ssems[(QB - 1) % NS_R]).wait()
        plsc.subcore_barrier()
        if stage:
            oa, ob = out_ref

            @pl.when(c == 0)
            def _():
                pltpu.sync_copy(acc.at[pl.ds(s * RS, RS)],
                                oa.at[pl.ds(s * RS, RS)])

            @pl.when(c == 1)
            def _():
                pltpu.sync_copy(acc.at[pl.ds(s * RS, RS)],
                                ob.at[pl.ds(s * RS, RS)])
        else:
            pltpu.sync_copy(acc.at[pl.ds(s * RS, RS)],
                            out_ref.at[pl.ds(c * NPAD + s * RS, RS)])
        plsc.subcore_barrier()


def _sc_aggregate(hs, srcs, dsts, npws, hd):
    z = jnp.zeros((NPAD, hd), jnp.float32)
    stage = hd * (N + NPAD) * 4 <= 6 * 2**20
    ns_r = 4 if stage else 2
    if stage:
        out_t = [jax.ShapeDtypeStruct((NPAD, hd), jnp.float32)] * 6
    else:
        out_t = [jax.ShapeDtypeStruct((2 * NPAD, hd), jnp.float32)] * 3
    scratch = [
        pltpu.VMEM_SHARED((NPAD, hd), jnp.float32),
        pltpu.VMEM((2, QB, CH), jnp.int32),
        pltpu.VMEM((2, QB, CH), jnp.int32),
        pltpu.VMEM((ns_r, CH, hd), jnp.float32),
    ] + [pltpu.SemaphoreType.DMA] * (2 * ns_r + 1)
    if stage:
        scratch.append(pltpu.VMEM_SHARED((N, hd), jnp.float32))
    fn = pl.kernel(
        functools.partial(_agg_body, tuple(npws), hd, stage, ns_r),
        out_type=out_t,
        mesh=_mesh(),
        scratch_types=scratch,
        compiler_params=pltpu.CompilerParams(use_tc_tiling_on_sc=(hd == H)),
        name=f"sc_gcn_agg_{hd}",
    )
    return fn(*hs, *srcs, *dsts, z)


_BLK = 2000
_G = N // _BLK
_DOT = dict(preferred_element_type=jnp.float32)


def _tc1a_body(*refs):
    for b in range(3):
        x_ref, w_ref = refs[2 * b:2 * b + 2]
        refs[6 + b][...] = lax.dot_general(
            x_ref[...], w_ref[...], (((1,), (0,)), ((), ())), **_DOT)


def _tc1a(xs, w1s):
    in_arrays, in_specs = [], []
    for x, w1 in zip(xs, w1s):
        d = x.shape[1]
        in_arrays += [x, w1]
        in_specs += [
            pl.BlockSpec((_BLK, d), lambda i: (i, 0)),
            pl.BlockSpec((d, H), lambda i: (0, 0)),
        ]
    return pl.pallas_call(
        _tc1a_body,
        grid=(_G,),
        in_specs=in_specs,
        out_specs=[pl.BlockSpec((_BLK, H), lambda i: (i, 0))] * 3,
        out_shape=[jax.ShapeDtypeStruct((N, H), jnp.float32)] * 3,
    )(*in_arrays)


def _tc1b_body(*refs):
    for b in range(3):
        h_ref, degp_ref = refs[2 * b:2 * b + 2]
        hp_ref, dinv_ref = refs[6 + 2 * b:6 + 2 * b + 2]
        deg = degp_ref[0, :, 0:1] + degp_ref[1, :, 0:1] + 1.0
        dinv = lax.rsqrt(deg)
        hp_ref[...] = h_ref[...] * dinv
        dinv_ref[...] = dinv


def _tc1b(hs, degps):
    in_arrays, in_specs = [], []
    for h, degp in zip(hs, degps):
        in_arrays += [h, degp]
        in_specs += [
            pl.BlockSpec((_BLK, H), lambda i: (i, 0)),
            pl.BlockSpec((2, _BLK, DW), lambda i: (0, i, 0)),
        ]
    return pl.pallas_call(
        _tc1b_body,
        grid=(_G,),
        in_specs=in_specs,
        out_specs=[
            pl.BlockSpec((_BLK, H), lambda i: (i, 0)),
            pl.BlockSpec((_BLK, 1), lambda i: (i, 0)),
        ] * 3,
        out_shape=[
            jax.ShapeDtypeStruct((N, H), jnp.float32),
            jax.ShapeDtypeStruct((N, 1), jnp.float32),
        ] * 3,
    )(*in_arrays)


def _tc2_body(*refs):
    for b in range(3):
        sp_ref, hp_ref, dinv_ref, b1_ref, w2_ref = refs[5 * b:5 * b + 5]
        out_ref = refs[15 + b]
        dinv = dinv_ref[...]
        y = (sp_ref[0] + sp_ref[1] + hp_ref[...]) * dinv + b1_ref[...]
        y = jnp.maximum(y, 0.0)
        h2 = lax.dot_general(y, w2_ref[...], (((1,), (0,)), ((), ())), **_DOT)
        out_ref[...] = h2 * dinv


def _tc2(sps, hps, dinvs, b1s, w2s):
    in_arrays, in_specs = [], []
    for sp, hp, dinv, b1, w2 in zip(sps, hps, dinvs, b1s, w2s):
        in_arrays += [sp, hp, dinv, b1.reshape(1, H), w2]
        in_specs += [
            pl.BlockSpec((2, _BLK, H), lambda i: (0, i, 0)),
            pl.BlockSpec((_BLK, H), lambda i: (i, 0)),
            pl.BlockSpec((_BLK, 1), lambda i: (i, 0)),
            pl.BlockSpec((1, H), lambda i: (0, 0)),
            pl.BlockSpec((H, OUT), lambda i: (0, 0)),
        ]
    return pl.pallas_call(
        _tc2_body,
        grid=(_G,),
        in_specs=in_specs,
        out_specs=[pl.BlockSpec((_BLK, OUT), lambda i: (i, 0))] * 3,
        out_shape=[jax.ShapeDtypeStruct((N, OUT), jnp.float32)] * 3,
    )(*in_arrays)


def _tc3_body(*refs):
    out_ref, pooled, counts = refs[-3:]
    demo_ref, f1w, f1b, f2w, f2b, f3w, f3b = refs[24:31]
    i = pl.program_id(0)

    @pl.when(i == 0)
    def _():
        pooled[...] = jnp.zeros_like(pooled)
        counts[...] = jnp.zeros_like(counts)

    hb = _BLK // 2
    ones_col = jnp.ones((hb, 1), jnp.float32)
    gids = lax.broadcasted_iota(jnp.int32, (B, hb), 0)
    for b in range(3):
        spa, spb, hp, die, dio, bb, bte, bto = refs[8 * b:8 * b + 8]
        yp = spa[...] + spb[...] + hp[...]
        ye = jnp.maximum(yp[:, :OUT] * die[...] + bb[...], 0.0)
        yo = jnp.maximum(yp[:, OUT:] * dio[...] + bb[...], 0.0)
        me = (gids == bte[0]).astype(jnp.float32)
        mo = (gids == bto[0]).astype(jnp.float32)
        pooled[:, b * OUT:(b + 1) * OUT] += (
            lax.dot_general(me, ye, (((1,), (0,)), ((), ())), **_DOT)
            + lax.dot_general(mo, yo, (((1,), (0,)), ((), ())), **_DOT))
        counts[:, b:b + 1] += lax.dot_general(
            me + mo, ones_col, (((1,), (0,)), ((), ())), **_DOT)

    @pl.when(i == _G - 1)
    def _():
        cnt = jnp.maximum(counts[...], 1.0)
        h = f1b[...]
        for b in range(3):
            p = pooled[:, b * OUT:(b + 1) * OUT] / cnt[:, b:b + 1]
            h = h + lax.dot_general(
                p, f1w[b * OUT:(b + 1) * OUT, :],
                (((1,), (0,)), ((), ())), **_DOT)
        h = h + lax.dot_general(demo_ref[...], f1w[3 * OUT:, :],
                                (((1,), (0,)), ((), ())), **_DOT)
        h = jnp.maximum(h, 0.0)
        h = jnp.maximum(lax.dot_general(h, f2w[...],
                                        (((1,), (0,)), ((), ())), **_DOT)
                        + f2b[...], 0.0)
        out_ref[...] = lax.dot_general(h, f3w[...],
                                       (((1,), (0,)), ((), ())), **_DOT) \
            + f3b[...]


def _tc3(branches, demo, f1w, f1b, f2w, f2b, f3w, f3b):
    hb = _BLK // 2
    in_arrays = []
    in_specs = []
    for spa, spb, hp, die, dio, b2, bte, bto in branches:
        in_arrays += [spa, spb, hp, die, dio, b2, bte, bto]
        in_specs += [
            pl.BlockSpec((hb, 2 * OUT), lambda i: (i, 0)),
            pl.BlockSpec((hb, 2 * OUT), lambda i: (i, 0)),
            pl.BlockSpec((hb, 2 * OUT), lambda i: (i, 0)),
            pl.BlockSpec((hb, 1), lambda i: (i, 0)),
            pl.BlockSpec((hb, 1), lambda i: (i, 0)),
            pl.BlockSpec((1, OUT), lambda i: (0, 0)),
            pl.BlockSpec((1, 1, hb), lambda i: (i, 0, 0)),
            pl.BlockSpec((1, 1, hb), lambda i: (i, 0, 0)),
        ]
    in_arrays += [demo, f1w, f1b, f2w, f2b, f3w, f3b]
    in_specs += [
        pl.BlockSpec((B, 16), lambda i: (0, 0)),
        pl.BlockSpec((3 * OUT + 16, B), lambda i: (0, 0)),
        pl.BlockSpec((1, B), lambda i: (0, 0)),
        pl.BlockSpec((B, 32), lambda i: (0, 0)),
        pl.BlockSpec((1, 32), lambda i: (0, 0)),
        pl.BlockSpec((32, 2), lambda i: (0, 0)),
        pl.BlockSpec((1, 2), lambda i: (0, 0)),
    ]
    return pl.pallas_call(
        _tc3_body,
        grid=(_G,),
        in_specs=in_specs,
        out_specs=pl.BlockSpec((B, 2), lambda i: (0, 0)),
        out_shape=jax.ShapeDtypeStruct((B, 2), jnp.float32),
        scratch_shapes=[
            pltpu.VMEM((B, 3 * OUT), jnp.float32),
            pltpu.VMEM((B, 8), jnp.float32),
        ],
    )(*in_arrays)


def kernel(x_desikan, edge_index_desikan, batch_desikan,
           x_destrieux, edge_index_destrieux, batch_destrieux,
           x_fuzzy, edge_index_fuzzy, batch_fuzzy,
           demographic,
           W1_des, b1_des, W2_des, b2_des,
           W1_det, b1_det, W2_det, b2_det,
           W1_fuz, b1_fuz, W2_fuz, b2_fuz,
           fc1_W, fc1_b, fc2_W, fc2_b, fc3_W, fc3_b):
    xs = (x_desikan, x_destrieux, x_fuzzy)
    eis = (edge_index_desikan, edge_index_destrieux, edge_index_fuzzy)
    bats = (batch_desikan, batch_destrieux, batch_fuzzy)
    w1s, b1s = (W1_des, W1_det, W1_fuz), (b1_des, b1_det, b1_fuz)
    w2s, b2s = (W2_des, W2_det, W2_fuz), (b2_des, b2_det, b2_fuz)

    e_pads = [-(-ei.shape[1] // E_ALIGN) * E_ALIGN for ei in eis]
    npws = tuple(e_pad // (NW * CH) for e_pad in e_pads)
    srcs, dsts = _edge_prep(eis, e_pads)
    srcs, dsts = tuple(srcs), tuple(dsts)

    degps = _sc_degrees(dsts, npws)
    degps = [p.reshape(2, NPAD, DW) for p in degps]

    h1s = _tc1a(xs, w1s)
    tc1_out = _tc1b(h1s, degps)
    h1ps = [tc1_out[0], tc1_out[2], tc1_out[4]]
    dinvs = [tc1_out[1], tc1_out[3], tc1_out[5]]

    s1ps = _sc_aggregate(tuple(h1ps), srcs, dsts, npws, H)
    s1ps = [p.reshape(2, NPAD, H) for p in s1ps]

    h2ps = _tc2(s1ps, h1ps, dinvs, b1s, w2s)

    s2ps = _sc_aggregate(tuple(h2ps), srcs, dsts, npws, OUT)

    hb = _BLK // 2
    branches = []
    for b, (hp, dinv, b2, bat) in enumerate(zip(h2ps, dinvs, b2s, bats)):
        spa = s2ps[2 * b].reshape(NPAD // 2, 2 * OUT)
        spb = s2ps[2 * b + 1].reshape(NPAD // 2, 2 * OUT)
        branches.append((spa, spb, hp.reshape(N // 2, 2 * OUT),
                         dinv[0::2], dinv[1::2], b2.reshape(1, OUT),
                         bat[0::2].reshape(_G, 1, hb),
                         bat[1::2].reshape(_G, 1, hb)))

    return _tc3(branches, demographic, fc1_W, fc1_b.reshape(1, B),
                fc2_W, fc2_b.reshape(1, 32), fc3_W, fc3_b.reshape(1, 2))
